# baseline (speedup 1.0000x reference)
import functools

import jax
import jax.numpy as jnp
from jax import lax
from jax.experimental import pallas as pl
from jax.experimental.pallas import tpu as pltpu, tpu_sc as plsc

N = 10000
E = 160000
D_IN = 128
H = 128
R = 2
B = 8
EDIM = 16
L = 3

NC, NS, LANES = 2, 16, 16
CH = 128
NCHUNK = 80
EPT = NCHUNK * CH
EPAD = NS * EPT
NPAD = 10240
RPT = NPAD // NS
HE = H + LANES

_sc_mesh = plsc.VectorSubcoreMesh(core_axis_name="c", subcore_axis_name="s")


GCH = 64
NBUF = 4
NGRPS = EPT // (NBUF * GCH)


@functools.partial(
    pl.kernel,
    out_type=jax.ShapeDtypeStruct((R, NPAD, H), jnp.float32),
    mesh=_sc_mesh,
    scratch_types=[
        pltpu.VMEM((2, NBUF, GCH), jnp.int32),
        pltpu.VMEM((2, NBUF, GCH), jnp.int32),
    ] + [pltpu.VMEM((GCH, H), jnp.float32)] * NBUF
      + [pltpu.VMEM_SHARED((NPAD, H), jnp.float32)]
      + [pltpu.SemaphoreType.DMA] * (2 * NBUF + 4),
)
def _sc_segment_sum(h_hbm, src_hbm, dst_hbm, out_hbm,
                    srcb, dstb, *rest):
    rows = rest[:NBUF]
    acc_sh = rest[NBUF]
    gsem = rest[NBUF + 1:2 * NBUF + 1]
    ssem = rest[2 * NBUF + 1:3 * NBUF + 1]
    isem = rest[3 * NBUF + 1:3 * NBUF + 3]
    jsem = rest[3 * NBUF + 3:3 * NBUF + 5]
    c = lax.axis_index("c")
    s = lax.axis_index("s")
    zero16 = jnp.full((LANES,), 0.0, jnp.float32)

    def zero_body(e, carry):
        for g in range(H // LANES):
            rows[0][e, pl.ds(g * LANES, LANES)] = zero16
        return carry

    lax.fori_loop(0, GCH, zero_body, 0)
    for q in range(RPT // GCH):
        pltpu.sync_copy(rows[0], acc_sh.at[pl.ds(s * RPT + q * GCH, GCH)])
    for p in range(2):
        pltpu.async_copy(src_hbm.at[c, s, p], srcb.at[p], isem[p])
        pltpu.async_copy(dst_hbm.at[c, s, p], dstb.at[p], jsem[p])
    plsc.subcore_barrier()

    def body(jj, carry):
        for p in range(2):
            g = jj * 2 + p
            pltpu.make_async_copy(src_hbm.at[c, s, g], srcb.at[p],
                                  isem[p]).wait()
            pltpu.make_async_copy(dst_hbm.at[c, s, g], dstb.at[p],
                                  jsem[p]).wait()
            gd = [pltpu.async_copy(h_hbm.at[srcb.at[p, b]], rows[b], gsem[b])
                  for b in range(NBUF)]
            sd = []
            for b in range(NBUF):
                gd[b].wait()
                sd.append(pltpu.async_copy(rows[b], acc_sh.at[dstb.at[p, b]],
                                           ssem[b], add=True))
            for d in sd:
                d.wait()
            nxt = g + 2

            @pl.when(nxt < NGRPS)
            def _():
                pltpu.async_copy(src_hbm.at[c, s, nxt], srcb.at[p], isem[p])
                pltpu.async_copy(dst_hbm.at[c, s, nxt], dstb.at[p], jsem[p])
        return carry

    lax.fori_loop(0, NGRPS // 2, body, 0)
    plsc.subcore_barrier()
    pltpu.sync_copy(acc_sh.at[pl.ds(s * RPT, RPT)],
                    out_hbm.at[c, pl.ds(s * RPT, RPT)])


@functools.partial(
    pl.kernel,
    out_type=jax.ShapeDtypeStruct((R, NPAD, H), jnp.float32),
    mesh=_sc_mesh,
    scratch_types=[
        pltpu.VMEM((NCHUNK, CH), jnp.int32),
        pltpu.VMEM((NCHUNK * CH,), jnp.float32),
        pltpu.VMEM((CH, H), jnp.float32),
        pltpu.VMEM((H,), jnp.float32),
        pltpu.VMEM((H,), jnp.float32),
        pltpu.VMEM_SHARED((NPAD, H), jnp.float32),
        pltpu.SemaphoreType.DMA,
    ],
)
def _sc_enh(base_hbm, wvec_hbm, w_hbm, dst_hbm, enh_hbm,
            dst_v, w_v, rows_v, base_v, wvec_v, acc_sh, sem):
    c = lax.axis_index("c")
    s = lax.axis_index("s")
    zero16 = jnp.full((LANES,), 0.0, jnp.float32)

    def zero_body(e, carry):
        for g in range(H // LANES):
            rows_v[e, pl.ds(g * LANES, LANES)] = zero16
        return carry

    lax.fori_loop(0, CH, zero_body, 0)
    for q in range(RPT // CH):
        pltpu.sync_copy(rows_v, acc_sh.at[pl.ds(s * RPT + q * CH, CH)])
    pltpu.sync_copy(base_hbm.at[c], base_v)
    pltpu.sync_copy(wvec_hbm, wvec_v)
    pltpu.sync_copy(dst_hbm.at[c, s], dst_v)
    pltpu.sync_copy(w_hbm.at[c, s], w_v)

    base_g = [base_v[pl.ds(g * LANES, LANES)] for g in range(H // LANES)]
    wvec_g = [wvec_v[pl.ds(g * LANES, LANES)] for g in range(H // LANES)]
    plsc.subcore_barrier()

    def chunk_body(j, carry):
        jbase = j * CH

        def grp_body(eb, carry2):
            wv16 = w_v[pl.ds(jbase + eb * LANES, LANES)]
            e0 = eb * LANES
            for k in range(LANES):
                wk = jnp.full((LANES,), wv16[k])
                for g in range(H // LANES):
                    rows_v[e0 + k, pl.ds(g * LANES, LANES)] = jnp.maximum(
                        base_g[g] + wk * wvec_g[g], 0.0)
            return carry2

        lax.fori_loop(0, CH // LANES, grp_body, 0)
        pltpu.sync_copy(rows_v, acc_sh.at[dst_v.at[j]], add=True)
        return carry

    lax.fori_loop(0, NCHUNK, chunk_body, 0)
    plsc.subcore_barrier()
    pltpu.sync_copy(acc_sh.at[pl.ds(s * RPT, RPT)],
                    enh_hbm.at[c, pl.ds(s * RPT, RPT)])


@functools.partial(
    pl.kernel,
    out_type=jax.ShapeDtypeStruct((R, NPAD, H), jnp.float32),
    mesh=_sc_mesh,
    scratch_types=[
        pltpu.VMEM((NCHUNK, CH), jnp.int32),
        pltpu.VMEM((CH, H), jnp.float32),
        pltpu.VMEM_SHARED((NPAD, H), jnp.float32),
        pltpu.SemaphoreType.DMA,
    ],
)
def _sc_counts(dst_hbm, cnt_hbm, dst_v, ones_v, acc_sh, sem):
    c = lax.axis_index("c")
    s = lax.axis_index("s")
    zero16 = jnp.full((LANES,), 0.0, jnp.float32)
    one0 = jnp.where(lax.iota(jnp.int32, LANES) == 0,
                     jnp.full((LANES,), 1.0, jnp.float32),
                     zero16)

    def zero_body(e, carry):
        for g in range(H // LANES):
            ones_v[e, pl.ds(g * LANES, LANES)] = zero16
        return carry

    lax.fori_loop(0, CH, zero_body, 0)
    for q in range(RPT // CH):
        pltpu.sync_copy(ones_v, acc_sh.at[pl.ds(s * RPT + q * CH, CH)])

    def ones_body(e, carry):
        ones_v[e, pl.ds(0, LANES)] = one0
        return carry

    lax.fori_loop(0, CH, ones_body, 0)
    pltpu.sync_copy(dst_hbm.at[c, s], dst_v)
    plsc.subcore_barrier()

    def cnt_body(j, carry):
        pltpu.sync_copy(ones_v, acc_sh.at[dst_v.at[j]], add=True)
        return carry

    lax.fori_loop(0, NCHUNK, cnt_body, 0)
    plsc.subcore_barrier()
    pltpu.sync_copy(acc_sh.at[pl.ds(s * RPT, RPT)],
                    cnt_hbm.at[c, pl.ds(s * RPT, RPT)])


MGCH = 32
MNBUF = 2
MNGRP = EPT // (MNBUF * MGCH)
NHALF = 5120
ACCR = 5128
ARPT = 320
SCRAPM = 5120


def _make_halfseg(k):
    @functools.partial(
        pl.kernel,
        out_type=jax.ShapeDtypeStruct((R, ACCR, H), jnp.float32),
        mesh=_sc_mesh,
        scratch_types=[
            pltpu.VMEM((2, MNBUF, MGCH), jnp.int32),
            pltpu.VMEM((2, MNBUF, MGCH), jnp.int32),
            pltpu.VMEM((MNBUF, MGCH), jnp.int32),
            pltpu.VMEM((MGCH,), jnp.int32),
            pltpu.VMEM((LANES,), jnp.int32),
        ] + [pltpu.VMEM((MGCH, H), jnp.float32)] * MNBUF
          + [pltpu.VMEM_SHARED((N, H), jnp.float32)]
          + [pltpu.VMEM_SHARED((ACCR, H), jnp.float32)]
          + [pltpu.SemaphoreType.DMA] * (2 * MNBUF + 4),
    )
    def halfseg(h_hbm, src_hbm, dst_hbm, cnt_hbm, out_hbm,
                srcb, dstb, dclamp, scrap_v, cnt_v, *rest):
        rows = rest[:MNBUF]
        mir_sh = rest[MNBUF]
        acc_sh = rest[MNBUF + 1]
        gsem = rest[MNBUF + 2:2 * MNBUF + 2]
        ssem = rest[2 * MNBUF + 2:2 * MNBUF + 4]
        isem = rest[2 * MNBUF + 4:2 * MNBUF + 6]
        jsem = rest[2 * MNBUF + 6:2 * MNBUF + 8]
        c = lax.axis_index("c")
        s = lax.axis_index("s")
        zero16 = jnp.full((LANES,), 0.0, jnp.float32)
        scrap16 = jnp.full((LANES,), SCRAPM, jnp.int32)
        half16 = jnp.full((LANES,), NHALF, jnp.int32)

        def zero_body(e, carry):
            for g in range(H // LANES):
                rows[0][e, pl.ds(g * LANES, LANES)] = zero16
            return carry

        lax.fori_loop(0, MGCH, zero_body, 0)
        for kk in range(MGCH // LANES):
            scrap_v[pl.ds(kk * LANES, LANES)] = scrap16
        for q in range(ARPT // MGCH):
            pltpu.sync_copy(rows[0],
                            acc_sh.at[pl.ds(s * ARPT + q * MGCH, MGCH)])

        @pl.when(s == NS - 1)
        def _():
            pltpu.sync_copy(rows[0].at[pl.ds(0, ACCR - NHALF)],
                            acc_sh.at[pl.ds(NHALF, ACCR - NHALF)])
        @pl.when(s < NS - 1)
        def _():
            pltpu.sync_copy(h_hbm.at[pl.ds(s * RPT, RPT)],
                            mir_sh.at[pl.ds(s * RPT, RPT)])

        @pl.when(s == NS - 1)
        def _():
            pltpu.sync_copy(h_hbm.at[pl.ds((NS - 1) * RPT, N - (NS - 1) * RPT)],
                            mir_sh.at[pl.ds((NS - 1) * RPT, N - (NS - 1) * RPT)])

        pltpu.sync_copy(cnt_hbm.at[c], cnt_v)
        cnt0 = cnt_v[pl.ds(0, LANES)][0]
        l1 = jnp.clip(cnt0 - s * EPT, 0, EPT)
        grpsz = MNBUF * MGCH
        if k == 0:
            glo = jnp.int32(0)
            ghi = (l1 + grpsz - 1) // grpsz
        else:
            glo = l1 // grpsz
            ghi = jnp.int32(MNGRP)
        n_pairs = (ghi - glo + 1) // 2
        for p in range(2):
            @pl.when(glo + p < ghi)
            def _(p=p):
                pltpu.async_copy(src_hbm.at[c, s, glo + p], srcb.at[p],
                                 isem[p])
                pltpu.async_copy(dst_hbm.at[c, s, glo + p], dstb.at[p],
                                 jsem[p])
        plsc.subcore_barrier()
        pltpu.async_copy(rows[0], acc_sh.at[scrap_v], ssem[0], add=True)
        pltpu.async_copy(rows[1], acc_sh.at[scrap_v], ssem[1], add=True)

        def body(jj, carry):
            for p in range(2):
                g = glo + jj * 2 + p

                @pl.when(g < ghi)
                def _(p=p, g=g):
                    pltpu.make_async_copy(src_hbm.at[c, s, g], srcb.at[p],
                                          isem[p]).wait()
                    pltpu.make_async_copy(dst_hbm.at[c, s, g], dstb.at[p],
                                          jsem[p]).wait()
                    gd = [pltpu.async_copy(mir_sh.at[srcb.at[p, b]], rows[b],
                                           gsem[b]) for b in range(MNBUF)]
                    for b in range(MNBUF):
                        gd[b].wait()
                        pltpu.make_async_copy(rows[b], acc_sh.at[scrap_v],
                                              ssem[b]).wait()
                        for v in range(MGCH // LANES):
                            dv = dstb[p, b, pl.ds(v * LANES, LANES)]
                            if k == 0:
                                d = jnp.where(dv < half16, dv, scrap16)
                            else:
                                d = jnp.where(dv >= half16, dv - half16,
                                              scrap16)
                            dclamp[b, pl.ds(v * LANES, LANES)] = d
                        pltpu.async_copy(rows[b], acc_sh.at[dclamp.at[b]],
                                         ssem[b], add=True)
                    nxt = g + 2

                    @pl.when(nxt < ghi)
                    def _():
                        pltpu.async_copy(src_hbm.at[c, s, nxt], srcb.at[p],
                                         isem[p])
                        pltpu.async_copy(dst_hbm.at[c, s, nxt], dstb.at[p],
                                         jsem[p])
            return carry

        lax.fori_loop(0, n_pairs, body, 0)
        for b in range(MNBUF):
            pltpu.make_async_copy(rows[b], acc_sh.at[scrap_v], ssem[b]).wait()
        plsc.subcore_barrier()
        pltpu.sync_copy(acc_sh.at[pl.ds(s * ARPT, ARPT)],
                        out_hbm.at[c, pl.ds(s * ARPT, ARPT)])

        @pl.when(s == NS - 1)
        def _():
            pltpu.sync_copy(acc_sh.at[pl.ds(NHALF, ACCR - NHALF)],
                            out_hbm.at[c, pl.ds(NHALF, ACCR - NHALF)])

    return halfseg


_halfseg0 = _make_halfseg(0)
_halfseg1 = _make_halfseg(1)


def _ln(y, g, b):
    mu = jnp.mean(y, axis=-1, keepdims=True)
    var = jnp.mean((y - mu) ** 2, axis=-1, keepdims=True)
    return (y - mu) * lax.rsqrt(var + 1e-5) * g + b


def _proj_body(x_ref, w_ref, b_ref, g_ref, bb_ref, out_ref):
    y = jnp.dot(x_ref[...], w_ref[...], preferred_element_type=jnp.float32)
    y = y + b_ref[...]
    out_ref[...] = jnp.maximum(_ln(y, g_ref[...], bb_ref[...]), 0.0)


def _base_body(emb_ref, w_ref, b_ref, out_ref):
    out_ref[...] = jnp.dot(emb_ref[...], w_ref[...],
                           preferred_element_type=jnp.float32) + b_ref[...]


def _wcomb_body(comp_ref, basis_ref, out_ref):
    out_ref[0] = jnp.dot(comp_ref[0], basis_ref[0],
                         preferred_element_type=jnp.float32)


def _layer_body(h_ref, slo_ref, shi_ref, enh_ref, cnt_ref, w_ref, root_ref,
                bias_ref, g_ref, b_ref, out_ref):
    c0 = cnt_ref[0]
    c1 = cnt_ref[1]
    deg = jnp.maximum(c0 + c1, 1.0)
    es = 0.1 * (enh_ref[0] + enh_ref[1]) / deg
    use_lo = pl.program_id(0) < NHALF // _ROWB
    s0 = jnp.where(use_lo, slo_ref[0], shi_ref[0])
    s1 = jnp.where(use_lo, slo_ref[1], shi_ref[1])
    m0 = s0 / jnp.maximum(c0, 1.0)
    m1 = s1 / jnp.maximum(c1, 1.0)
    h = h_ref[...]
    agg = (jnp.dot(m0, w_ref[0, 0], preferred_element_type=jnp.float32)
           + jnp.dot(m1, w_ref[0, 1], preferred_element_type=jnp.float32)
           + jnp.dot(h, root_ref[0], preferred_element_type=jnp.float32)
           + bias_ref[0] + es)
    out_ref[...] = jnp.maximum(_ln(agg, g_ref[0], b_ref[0]), 0.0) + h


_ROWB = 1024
_NBLK = (N + _ROWB - 1) // _ROWB


def _tc_proj(x, w, b, g, bb):
    return pl.pallas_call(
        _proj_body,
        grid=(_NBLK,),
        in_specs=[
            pl.BlockSpec((_ROWB, D_IN), lambda i: (i, 0)),
            pl.BlockSpec((D_IN, H), lambda i: (0, 0)),
            pl.BlockSpec((1, H), lambda i: (0, 0)),
            pl.BlockSpec((1, H), lambda i: (0, 0)),
            pl.BlockSpec((1, H), lambda i: (0, 0)),
        ],
        out_specs=pl.BlockSpec((_ROWB, H), lambda i: (i, 0)),
        out_shape=jax.ShapeDtypeStruct((N, H), jnp.float32),
    )(x, w, b, g, bb)


def _tc_base(emb, w16, b):
    return pl.pallas_call(
        _base_body,
        out_shape=jax.ShapeDtypeStruct((R, H), jnp.float32),
    )(emb, w16, b)


def _tc_wcomb(comp, basis_r):
    return pl.pallas_call(
        _wcomb_body,
        grid=(L,),
        in_specs=[
            pl.BlockSpec((1, R, B), lambda l: (l, 0, 0)),
            pl.BlockSpec((1, B, H * H), lambda l: (l, 0, 0)),
        ],
        out_specs=pl.BlockSpec((1, R, H * H), lambda l: (l, 0, 0)),
        out_shape=jax.ShapeDtypeStruct((L, R, H * H), jnp.float32),
    )(comp, basis_r)


def _tc_layer(l, h, s_lo, s_hi, enh, cnt, w3, root, bias, ng, nb):
    nlo = NHALF // _ROWB
    return pl.pallas_call(
        functools.partial(_layer_body),
        grid=(_NBLK,),
        in_specs=[
            pl.BlockSpec((_ROWB, H), lambda i: (i, 0)),
            pl.BlockSpec((R, _ROWB, H),
                         lambda i, _n=nlo: (0, jnp.minimum(i, _n - 1), 0)),
            pl.BlockSpec((R, _ROWB, H),
                         lambda i, _n=nlo: (0, jnp.maximum(i - _n, 0), 0)),
            pl.BlockSpec((R, _ROWB, H), lambda i: (0, i, 0)),
            pl.BlockSpec((R, _ROWB, 1), lambda i: (0, i, 0)),
            pl.BlockSpec((1, R, H, H), lambda i, _l=l: (_l, 0, 0, 0)),
            pl.BlockSpec((1, H, H), lambda i, _l=l: (_l, 0, 0)),
            pl.BlockSpec((1, 1, H), lambda i, _l=l: (_l, 0, 0)),
            pl.BlockSpec((1, 1, H), lambda i, _l=l: (_l, 0, 0)),
            pl.BlockSpec((1, 1, H), lambda i, _l=l: (_l, 0, 0)),
        ],
        out_specs=pl.BlockSpec((_ROWB, H), lambda i: (i, 0)),
        out_shape=jax.ShapeDtypeStruct((N, H), jnp.float32),
    )(h, s_lo, s_hi, enh, cnt, w3, root, bias, ng, nb)


def _partition_rel(src_r, dst_r):
    src_r = src_r.astype(jnp.int32)
    dst_r = dst_r.astype(jnp.int32)
    key = (dst_r >= NHALF).astype(jnp.int32)
    cnt0 = E - key.sum()
    pos = jnp.where(key == 0, jnp.cumsum(1 - key) - 1,
                    cnt0 + jnp.cumsum(key) - 1)
    sp = jnp.zeros((E,), jnp.int32).at[pos].set(src_r)
    dp = jnp.zeros((E,), jnp.int32).at[pos].set(dst_r)
    pad = EPAD - E
    sp = jnp.concatenate([sp, jnp.zeros((pad,), jnp.int32)])
    dp = jnp.concatenate([dp, jnp.full((pad,), 2 * NHALF, jnp.int32)])
    return sp, dp, cnt0


def _prep_rel(idx_row, pad_val, dtype):
    pad = EPAD - E
    arr = jnp.concatenate(
        [idx_row.astype(dtype), jnp.full((pad,), pad_val, dtype)])
    return arr.reshape(NS, NCHUNK, CH)


def kernel(x, edge_index_r0, edge_index_r1, edge_attr_r0, edge_attr_r1,
           proj_W, proj_b, proj_ln_g, proj_ln_b, edge_emb, emlp_W, emlp_b,
           conv_comp, conv_basis, conv_root, conv_bias, norm_g, norm_b):
    src = jnp.stack([_prep_rel(edge_index_r0[0], 0, jnp.int32),
                     _prep_rel(edge_index_r1[0], 0, jnp.int32)])
    dst = jnp.stack([_prep_rel(edge_index_r0[1], N, jnp.int32),
                     _prep_rel(edge_index_r1[1], N, jnp.int32)])
    sp0, dp0, cnt0_0 = _partition_rel(edge_index_r0[0], edge_index_r0[1])
    sp1, dp1, cnt0_1 = _partition_rel(edge_index_r1[0], edge_index_r1[1])
    src_m = jnp.stack([sp0, sp1]).reshape(R, NS, MNGRP, MNBUF, MGCH)
    dst_m = jnp.stack([dp0, dp1]).reshape(R, NS, MNGRP, MNBUF, MGCH)
    cnt_arr = jnp.broadcast_to(
        jnp.stack([cnt0_0, cnt0_1]).astype(jnp.int32)[:, None], (R, LANES))
    cnt_arr = cnt_arr + jnp.zeros((R, LANES), jnp.int32)
    wgt = jnp.stack([_prep_rel(edge_attr_r0[:, 1], 0.0, jnp.float32),
                     _prep_rel(edge_attr_r1[:, 1], 0.0, jnp.float32)]
                    ).reshape(R, NS, NCHUNK * CH)

    h = _tc_proj(x, proj_W, proj_b.reshape(1, H), proj_ln_g.reshape(1, H),
                 proj_ln_b.reshape(1, H))
    base = _tc_base(edge_emb, emlp_W[:EDIM], emlp_b.reshape(1, H))
    w3 = _tc_wcomb(conv_comp, conv_basis.reshape(L, B, H * H)
                   ).reshape(L, R, H, H)

    enh = _sc_enh(base, emlp_W[EDIM], wgt, dst)
    cnt_full = _sc_counts(dst)
    cnt = cnt_full[:, :, 0:1]

    for l in range(L):
        s_lo = _halfseg0(h, src_m, dst_m, cnt_arr)
        s_hi = _halfseg1(h, src_m, dst_m, cnt_arr)
        h = _tc_layer(l, h, s_lo, s_hi, enh, cnt, w3, conv_root,
                      conv_bias.reshape(L, 1, H), norm_g.reshape(L, 1, H),
                      norm_b.reshape(L, 1, H))
    return h


# speedup vs baseline: 2.3433x; 2.3433x over previous
import functools

import jax
import jax.numpy as jnp
from jax import lax
from jax.experimental import pallas as pl
from jax.experimental.pallas import tpu as pltpu, tpu_sc as plsc

N = 10000
E = 160000
D_IN = 128
H = 128
R = 2
B = 8
EDIM = 16
L = 3

NC, NS, LANES = 2, 16, 16
CH = 128
NCHUNK = 80
EPT = NCHUNK * CH
EPAD = NS * EPT
NPAD = 10240
RPT = NPAD // NS
HE = H + LANES

_sc_mesh = plsc.VectorSubcoreMesh(core_axis_name="c", subcore_axis_name="s")


GCH = 64
NBUF = 4
NGRPS = EPT // (NBUF * GCH)


@functools.partial(
    pl.kernel,
    out_type=jax.ShapeDtypeStruct((R, NPAD, H), jnp.float32),
    mesh=_sc_mesh,
    scratch_types=[
        pltpu.VMEM((2, NBUF, GCH), jnp.int32),
        pltpu.VMEM((2, NBUF, GCH), jnp.int32),
    ] + [pltpu.VMEM((GCH, H), jnp.float32)] * NBUF
      + [pltpu.VMEM_SHARED((NPAD, H), jnp.float32)]
      + [pltpu.SemaphoreType.DMA] * (2 * NBUF + 4),
)
def _sc_segment_sum(h_hbm, src_hbm, dst_hbm, out_hbm,
                    srcb, dstb, *rest):
    rows = rest[:NBUF]
    acc_sh = rest[NBUF]
    gsem = rest[NBUF + 1:2 * NBUF + 1]
    ssem = rest[2 * NBUF + 1:3 * NBUF + 1]
    isem = rest[3 * NBUF + 1:3 * NBUF + 3]
    jsem = rest[3 * NBUF + 3:3 * NBUF + 5]
    c = lax.axis_index("c")
    s = lax.axis_index("s")
    zero16 = jnp.full((LANES,), 0.0, jnp.float32)

    def zero_body(e, carry):
        for g in range(H // LANES):
            rows[0][e, pl.ds(g * LANES, LANES)] = zero16
        return carry

    lax.fori_loop(0, GCH, zero_body, 0)
    for q in range(RPT // GCH):
        pltpu.sync_copy(rows[0], acc_sh.at[pl.ds(s * RPT + q * GCH, GCH)])
    for p in range(2):
        pltpu.async_copy(src_hbm.at[c, s, p], srcb.at[p], isem[p])
        pltpu.async_copy(dst_hbm.at[c, s, p], dstb.at[p], jsem[p])
    plsc.subcore_barrier()

    def body(jj, carry):
        for p in range(2):
            g = jj * 2 + p
            pltpu.make_async_copy(src_hbm.at[c, s, g], srcb.at[p],
                                  isem[p]).wait()
            pltpu.make_async_copy(dst_hbm.at[c, s, g], dstb.at[p],
                                  jsem[p]).wait()
            gd = [pltpu.async_copy(h_hbm.at[srcb.at[p, b]], rows[b], gsem[b])
                  for b in range(NBUF)]
            sd = []
            for b in range(NBUF):
                gd[b].wait()
                sd.append(pltpu.async_copy(rows[b], acc_sh.at[dstb.at[p, b]],
                                           ssem[b], add=True))
            for d in sd:
                d.wait()
            nxt = g + 2

            @pl.when(nxt < NGRPS)
            def _():
                pltpu.async_copy(src_hbm.at[c, s, nxt], srcb.at[p], isem[p])
                pltpu.async_copy(dst_hbm.at[c, s, nxt], dstb.at[p], jsem[p])
        return carry

    lax.fori_loop(0, NGRPS // 2, body, 0)
    plsc.subcore_barrier()
    pltpu.sync_copy(acc_sh.at[pl.ds(s * RPT, RPT)],
                    out_hbm.at[c, pl.ds(s * RPT, RPT)])


@functools.partial(
    pl.kernel,
    out_type=jax.ShapeDtypeStruct((R, NPAD, H), jnp.float32),
    mesh=_sc_mesh,
    scratch_types=[
        pltpu.VMEM((NCHUNK, CH), jnp.int32),
        pltpu.VMEM((NCHUNK * CH,), jnp.float32),
        pltpu.VMEM((CH, H), jnp.float32),
        pltpu.VMEM((H,), jnp.float32),
        pltpu.VMEM((H,), jnp.float32),
        pltpu.VMEM_SHARED((NPAD, H), jnp.float32),
        pltpu.SemaphoreType.DMA,
    ],
)
def _sc_enh(base_hbm, wvec_hbm, w_hbm, dst_hbm, enh_hbm,
            dst_v, w_v, rows_v, base_v, wvec_v, acc_sh, sem):
    c = lax.axis_index("c")
    s = lax.axis_index("s")
    zero16 = jnp.full((LANES,), 0.0, jnp.float32)

    def zero_body(e, carry):
        for g in range(H // LANES):
            rows_v[e, pl.ds(g * LANES, LANES)] = zero16
        return carry

    lax.fori_loop(0, CH, zero_body, 0)
    for q in range(RPT // CH):
        pltpu.sync_copy(rows_v, acc_sh.at[pl.ds(s * RPT + q * CH, CH)])
    pltpu.sync_copy(base_hbm.at[c], base_v)
    pltpu.sync_copy(wvec_hbm, wvec_v)
    pltpu.sync_copy(dst_hbm.at[c, s], dst_v)
    pltpu.sync_copy(w_hbm.at[c, s], w_v)

    base_g = [base_v[pl.ds(g * LANES, LANES)] for g in range(H // LANES)]
    wvec_g = [wvec_v[pl.ds(g * LANES, LANES)] for g in range(H // LANES)]
    plsc.subcore_barrier()

    def chunk_body(j, carry):
        jbase = j * CH

        def grp_body(eb, carry2):
            wv16 = w_v[pl.ds(jbase + eb * LANES, LANES)]
            e0 = eb * LANES
            for k in range(LANES):
                wk = jnp.full((LANES,), wv16[k])
                for g in range(H // LANES):
                    rows_v[e0 + k, pl.ds(g * LANES, LANES)] = jnp.maximum(
                        base_g[g] + wk * wvec_g[g], 0.0)
            return carry2

        lax.fori_loop(0, CH // LANES, grp_body, 0)
        pltpu.sync_copy(rows_v, acc_sh.at[dst_v.at[j]], add=True)
        return carry

    lax.fori_loop(0, NCHUNK, chunk_body, 0)
    plsc.subcore_barrier()
    pltpu.sync_copy(acc_sh.at[pl.ds(s * RPT, RPT)],
                    enh_hbm.at[c, pl.ds(s * RPT, RPT)])


@functools.partial(
    pl.kernel,
    out_type=jax.ShapeDtypeStruct((R, NPAD, H), jnp.float32),
    mesh=_sc_mesh,
    scratch_types=[
        pltpu.VMEM((NCHUNK, CH), jnp.int32),
        pltpu.VMEM((CH, H), jnp.float32),
        pltpu.VMEM_SHARED((NPAD, H), jnp.float32),
        pltpu.SemaphoreType.DMA,
    ],
)
def _sc_counts(dst_hbm, cnt_hbm, dst_v, ones_v, acc_sh, sem):
    c = lax.axis_index("c")
    s = lax.axis_index("s")
    zero16 = jnp.full((LANES,), 0.0, jnp.float32)
    one0 = jnp.where(lax.iota(jnp.int32, LANES) == 0,
                     jnp.full((LANES,), 1.0, jnp.float32),
                     zero16)

    def zero_body(e, carry):
        for g in range(H // LANES):
            ones_v[e, pl.ds(g * LANES, LANES)] = zero16
        return carry

    lax.fori_loop(0, CH, zero_body, 0)
    for q in range(RPT // CH):
        pltpu.sync_copy(ones_v, acc_sh.at[pl.ds(s * RPT + q * CH, CH)])

    def ones_body(e, carry):
        ones_v[e, pl.ds(0, LANES)] = one0
        return carry

    lax.fori_loop(0, CH, ones_body, 0)
    pltpu.sync_copy(dst_hbm.at[c, s], dst_v)
    plsc.subcore_barrier()

    def cnt_body(j, carry):
        pltpu.sync_copy(ones_v, acc_sh.at[dst_v.at[j]], add=True)
        return carry

    lax.fori_loop(0, NCHUNK, cnt_body, 0)
    plsc.subcore_barrier()
    pltpu.sync_copy(acc_sh.at[pl.ds(s * RPT, RPT)],
                    cnt_hbm.at[c, pl.ds(s * RPT, RPT)])


MGCH = 32
MNBUF = 2
MNGRP = EPT // (MNBUF * MGCH)
NHALF = 5120
ACCR = 5128
ARPT = 320
SCRAPM = 5120


def _make_halfseg(k):
    @functools.partial(
        pl.kernel,
        out_type=jax.ShapeDtypeStruct((R, ACCR, H), jnp.float32),
        mesh=_sc_mesh,
        scratch_types=[
            pltpu.VMEM((2, MNBUF, MGCH), jnp.int32),
            pltpu.VMEM((2, MNBUF, MGCH), jnp.int32),
            pltpu.VMEM((MNBUF, MGCH), jnp.int32),
            pltpu.VMEM((MGCH,), jnp.int32),
            pltpu.VMEM((LANES,), jnp.int32),
        ] + [pltpu.VMEM((MGCH, H), jnp.float32)] * MNBUF
          + [pltpu.VMEM_SHARED((N, H), jnp.float32)]
          + [pltpu.VMEM_SHARED((ACCR, H), jnp.float32)]
          + [pltpu.SemaphoreType.DMA] * (2 * MNBUF + 4),
    )
    def halfseg(h_hbm, src_hbm, dst_hbm, cnt_hbm, out_hbm,
                srcb, dstb, dclamp, scrap_v, cnt_v, *rest):
        rows = rest[:MNBUF]
        mir_sh = rest[MNBUF]
        acc_sh = rest[MNBUF + 1]
        gsem = rest[MNBUF + 2:2 * MNBUF + 2]
        ssem = rest[2 * MNBUF + 2:2 * MNBUF + 4]
        isem = rest[2 * MNBUF + 4:2 * MNBUF + 6]
        jsem = rest[2 * MNBUF + 6:2 * MNBUF + 8]
        c = lax.axis_index("c")
        s = lax.axis_index("s")
        zero16 = jnp.full((LANES,), 0.0, jnp.float32)
        scrap16 = jnp.full((LANES,), SCRAPM, jnp.int32)
        half16 = jnp.full((LANES,), NHALF, jnp.int32)

        def zero_body(e, carry):
            for g in range(H // LANES):
                rows[0][e, pl.ds(g * LANES, LANES)] = zero16
            return carry

        lax.fori_loop(0, MGCH, zero_body, 0)
        for kk in range(MGCH // LANES):
            scrap_v[pl.ds(kk * LANES, LANES)] = scrap16
        for q in range(ARPT // MGCH):
            pltpu.sync_copy(rows[0],
                            acc_sh.at[pl.ds(s * ARPT + q * MGCH, MGCH)])

        @pl.when(s == NS - 1)
        def _():
            pltpu.sync_copy(rows[0].at[pl.ds(0, ACCR - NHALF)],
                            acc_sh.at[pl.ds(NHALF, ACCR - NHALF)])
        @pl.when(s < NS - 1)
        def _():
            pltpu.sync_copy(h_hbm.at[pl.ds(s * RPT, RPT)],
                            mir_sh.at[pl.ds(s * RPT, RPT)])

        @pl.when(s == NS - 1)
        def _():
            pltpu.sync_copy(h_hbm.at[pl.ds((NS - 1) * RPT, N - (NS - 1) * RPT)],
                            mir_sh.at[pl.ds((NS - 1) * RPT, N - (NS - 1) * RPT)])

        pltpu.sync_copy(cnt_hbm.at[c], cnt_v)
        cnt0 = cnt_v[pl.ds(0, LANES)][0]
        l1 = jnp.clip(cnt0 - s * EPT, 0, EPT)
        grpsz = MNBUF * MGCH
        if k == 0:
            glo = jnp.int32(0)
            ghi = (l1 + grpsz - 1) // grpsz
        else:
            glo = l1 // grpsz
            ghi = jnp.int32(MNGRP)
        n_pairs = (ghi - glo + 1) // 2
        for p in range(2):
            @pl.when(glo + p < ghi)
            def _(p=p):
                pltpu.async_copy(src_hbm.at[c, s, glo + p], srcb.at[p],
                                 isem[p])
                pltpu.async_copy(dst_hbm.at[c, s, glo + p], dstb.at[p],
                                 jsem[p])
        plsc.subcore_barrier()
        pltpu.async_copy(rows[0], acc_sh.at[scrap_v], ssem[0], add=True)
        pltpu.async_copy(rows[1], acc_sh.at[scrap_v], ssem[1], add=True)

        def body(jj, carry):
            for p in range(2):
                g = glo + jj * 2 + p

                @pl.when(g < ghi)
                def _(p=p, g=g):
                    pltpu.make_async_copy(src_hbm.at[c, s, g], srcb.at[p],
                                          isem[p]).wait()
                    pltpu.make_async_copy(dst_hbm.at[c, s, g], dstb.at[p],
                                          jsem[p]).wait()
                    gd = [pltpu.async_copy(mir_sh.at[srcb.at[p, b]], rows[b],
                                           gsem[b]) for b in range(MNBUF)]
                    for b in range(MNBUF):
                        gd[b].wait()
                        pltpu.make_async_copy(rows[b], acc_sh.at[scrap_v],
                                              ssem[b]).wait()
                        for v in range(MGCH // LANES):
                            dv = dstb[p, b, pl.ds(v * LANES, LANES)]
                            if k == 0:
                                d = jnp.where(dv < half16, dv, scrap16)
                            else:
                                d = jnp.where(dv >= half16, dv - half16,
                                              scrap16)
                            dclamp[b, pl.ds(v * LANES, LANES)] = d
                        pltpu.async_copy(rows[b], acc_sh.at[dclamp.at[b]],
                                         ssem[b], add=True)
                    nxt = g + 2

                    @pl.when(nxt < ghi)
                    def _():
                        pltpu.async_copy(src_hbm.at[c, s, nxt], srcb.at[p],
                                         isem[p])
                        pltpu.async_copy(dst_hbm.at[c, s, nxt], dstb.at[p],
                                         jsem[p])
            return carry

        lax.fori_loop(0, n_pairs, body, 0)
        for b in range(MNBUF):
            pltpu.make_async_copy(rows[b], acc_sh.at[scrap_v], ssem[b]).wait()
        plsc.subcore_barrier()
        pltpu.sync_copy(acc_sh.at[pl.ds(s * ARPT, ARPT)],
                        out_hbm.at[c, pl.ds(s * ARPT, ARPT)])

        @pl.when(s == NS - 1)
        def _():
            pltpu.sync_copy(acc_sh.at[pl.ds(NHALF, ACCR - NHALF)],
                            out_hbm.at[c, pl.ds(NHALF, ACCR - NHALF)])

    return halfseg


_halfseg0 = _make_halfseg(0)
_halfseg1 = _make_halfseg(1)


def _ln(y, g, b):
    mu = jnp.mean(y, axis=-1, keepdims=True)
    var = jnp.mean((y - mu) ** 2, axis=-1, keepdims=True)
    return (y - mu) * lax.rsqrt(var + 1e-5) * g + b


def _proj_body(x_ref, w_ref, b_ref, g_ref, bb_ref, out_ref):
    y = jnp.dot(x_ref[...], w_ref[...], preferred_element_type=jnp.float32)
    y = y + b_ref[...]
    out_ref[...] = jnp.maximum(_ln(y, g_ref[...], bb_ref[...]), 0.0)


def _base_body(emb_ref, w_ref, b_ref, out_ref):
    out_ref[...] = jnp.dot(emb_ref[...], w_ref[...],
                           preferred_element_type=jnp.float32) + b_ref[...]


def _wcomb_body(comp_ref, basis_ref, out_ref):
    out_ref[0] = jnp.dot(comp_ref[0], basis_ref[0],
                         preferred_element_type=jnp.float32)


def _layer_body(h_ref, slo_ref, shi_ref, enh_ref, cnt_ref, w_ref, root_ref,
                bias_ref, g_ref, b_ref, out_ref):
    c0 = cnt_ref[0]
    c1 = cnt_ref[1]
    deg = jnp.maximum(c0 + c1, 1.0)
    es = 0.1 * (enh_ref[0] + enh_ref[1]) / deg
    use_lo = pl.program_id(0) < NHALF // _ROWB
    s0 = jnp.where(use_lo, slo_ref[0], shi_ref[0])
    s1 = jnp.where(use_lo, slo_ref[1], shi_ref[1])
    m0 = s0 / jnp.maximum(c0, 1.0)
    m1 = s1 / jnp.maximum(c1, 1.0)
    h = h_ref[...]
    agg = (jnp.dot(m0, w_ref[0, 0], preferred_element_type=jnp.float32)
           + jnp.dot(m1, w_ref[0, 1], preferred_element_type=jnp.float32)
           + jnp.dot(h, root_ref[0], preferred_element_type=jnp.float32)
           + bias_ref[0] + es)
    out_ref[...] = jnp.maximum(_ln(agg, g_ref[0], b_ref[0]), 0.0) + h


_ROWB = 1024
_NBLK = (N + _ROWB - 1) // _ROWB


def _tc_proj(x, w, b, g, bb):
    return pl.pallas_call(
        _proj_body,
        grid=(_NBLK,),
        in_specs=[
            pl.BlockSpec((_ROWB, D_IN), lambda i: (i, 0)),
            pl.BlockSpec((D_IN, H), lambda i: (0, 0)),
            pl.BlockSpec((1, H), lambda i: (0, 0)),
            pl.BlockSpec((1, H), lambda i: (0, 0)),
            pl.BlockSpec((1, H), lambda i: (0, 0)),
        ],
        out_specs=pl.BlockSpec((_ROWB, H), lambda i: (i, 0)),
        out_shape=jax.ShapeDtypeStruct((N, H), jnp.float32),
    )(x, w, b, g, bb)


def _tc_base(emb, w16, b):
    return pl.pallas_call(
        _base_body,
        out_shape=jax.ShapeDtypeStruct((R, H), jnp.float32),
    )(emb, w16, b)


def _tc_wcomb(comp, basis_r):
    return pl.pallas_call(
        _wcomb_body,
        grid=(L,),
        in_specs=[
            pl.BlockSpec((1, R, B), lambda l: (l, 0, 0)),
            pl.BlockSpec((1, B, H * H), lambda l: (l, 0, 0)),
        ],
        out_specs=pl.BlockSpec((1, R, H * H), lambda l: (l, 0, 0)),
        out_shape=jax.ShapeDtypeStruct((L, R, H * H), jnp.float32),
    )(comp, basis_r)


def _tc_layer(l, h, s_lo, s_hi, enh, cnt, w3, root, bias, ng, nb):
    nlo = NHALF // _ROWB
    return pl.pallas_call(
        functools.partial(_layer_body),
        grid=(_NBLK,),
        in_specs=[
            pl.BlockSpec((_ROWB, H), lambda i: (i, 0)),
            pl.BlockSpec((R, _ROWB, H),
                         lambda i, _n=nlo: (0, jnp.minimum(i, _n - 1), 0)),
            pl.BlockSpec((R, _ROWB, H),
                         lambda i, _n=nlo: (0, jnp.maximum(i - _n, 0), 0)),
            pl.BlockSpec((R, _ROWB, H), lambda i: (0, i, 0)),
            pl.BlockSpec((R, _ROWB, 1), lambda i: (0, i, 0)),
            pl.BlockSpec((1, R, H, H), lambda i, _l=l: (_l, 0, 0, 0)),
            pl.BlockSpec((1, H, H), lambda i, _l=l: (_l, 0, 0)),
            pl.BlockSpec((1, 1, H), lambda i, _l=l: (_l, 0, 0)),
            pl.BlockSpec((1, 1, H), lambda i, _l=l: (_l, 0, 0)),
            pl.BlockSpec((1, 1, H), lambda i, _l=l: (_l, 0, 0)),
        ],
        out_specs=pl.BlockSpec((_ROWB, H), lambda i: (i, 0)),
        out_shape=jax.ShapeDtypeStruct((N, H), jnp.float32),
    )(h, s_lo, s_hi, enh, cnt, w3, root, bias, ng, nb)


def _partition_rel(src_r, dst_r):
    src_r = src_r.astype(jnp.int32)
    dst_r = dst_r.astype(jnp.int32)
    key = (dst_r >= NHALF).astype(jnp.int32)
    cnt0 = E - key.sum()
    key, dp, sp = jax.lax.sort((key, dst_r, src_r), num_keys=1,
                               is_stable=False)
    pad = EPAD - E
    sp = jnp.concatenate([sp, jnp.zeros((pad,), jnp.int32)])
    dp = jnp.concatenate([dp, jnp.full((pad,), 2 * NHALF, jnp.int32)])
    return sp, dp, cnt0


def _prep_rel(idx_row, pad_val, dtype):
    pad = EPAD - E
    arr = jnp.concatenate(
        [idx_row.astype(dtype), jnp.full((pad,), pad_val, dtype)])
    return arr.reshape(NS, NCHUNK, CH)


def kernel(x, edge_index_r0, edge_index_r1, edge_attr_r0, edge_attr_r1,
           proj_W, proj_b, proj_ln_g, proj_ln_b, edge_emb, emlp_W, emlp_b,
           conv_comp, conv_basis, conv_root, conv_bias, norm_g, norm_b):
    src = jnp.stack([_prep_rel(edge_index_r0[0], 0, jnp.int32),
                     _prep_rel(edge_index_r1[0], 0, jnp.int32)])
    dst = jnp.stack([_prep_rel(edge_index_r0[1], N, jnp.int32),
                     _prep_rel(edge_index_r1[1], N, jnp.int32)])
    sp0, dp0, cnt0_0 = _partition_rel(edge_index_r0[0], edge_index_r0[1])
    sp1, dp1, cnt0_1 = _partition_rel(edge_index_r1[0], edge_index_r1[1])
    src_m = jnp.stack([sp0, sp1]).reshape(R, NS, MNGRP, MNBUF, MGCH)
    dst_m = jnp.stack([dp0, dp1]).reshape(R, NS, MNGRP, MNBUF, MGCH)
    cnt_arr = jnp.broadcast_to(
        jnp.stack([cnt0_0, cnt0_1]).astype(jnp.int32)[:, None], (R, LANES))
    cnt_arr = cnt_arr + jnp.zeros((R, LANES), jnp.int32)
    wgt = jnp.stack([_prep_rel(edge_attr_r0[:, 1], 0.0, jnp.float32),
                     _prep_rel(edge_attr_r1[:, 1], 0.0, jnp.float32)]
                    ).reshape(R, NS, NCHUNK * CH)

    h = _tc_proj(x, proj_W, proj_b.reshape(1, H), proj_ln_g.reshape(1, H),
                 proj_ln_b.reshape(1, H))
    base = _tc_base(edge_emb, emlp_W[:EDIM], emlp_b.reshape(1, H))
    w3 = _tc_wcomb(conv_comp, conv_basis.reshape(L, B, H * H)
                   ).reshape(L, R, H, H)

    enh = _sc_enh(base, emlp_W[EDIM], wgt, dst)
    cnt_full = _sc_counts(dst)
    cnt = cnt_full[:, :, 0:1]

    for l in range(L):
        s_lo = _halfseg0(h, src_m, dst_m, cnt_arr)
        s_hi = _halfseg1(h, src_m, dst_m, cnt_arr)
        h = _tc_layer(l, h, s_lo, s_hi, enh, cnt, w3, conv_root,
                      conv_bias.reshape(L, 1, H), norm_g.reshape(L, 1, H),
                      norm_b.reshape(L, 1, H))
    return h


# speedup vs baseline: 2.7888x; 1.1901x over previous
import functools

import jax
import jax.numpy as jnp
from jax import lax
from jax.experimental import pallas as pl
from jax.experimental.pallas import tpu as pltpu, tpu_sc as plsc

N = 10000
E = 160000
D_IN = 128
H = 128
R = 2
B = 8
EDIM = 16
L = 3

NC, NS, LANES = 2, 16, 16
CH = 128
NCHUNK = 80
EPT = NCHUNK * CH
EPAD = NS * EPT
NPAD = 10240
RPT = NPAD // NS
HE = H + LANES

_sc_mesh = plsc.VectorSubcoreMesh(core_axis_name="c", subcore_axis_name="s")


GCH = 64
NBUF = 4
NGRPS = EPT // (NBUF * GCH)


@functools.partial(
    pl.kernel,
    out_type=jax.ShapeDtypeStruct((R, NPAD, H), jnp.float32),
    mesh=_sc_mesh,
    scratch_types=[
        pltpu.VMEM((2, NBUF, GCH), jnp.int32),
        pltpu.VMEM((2, NBUF, GCH), jnp.int32),
    ] + [pltpu.VMEM((GCH, H), jnp.float32)] * NBUF
      + [pltpu.VMEM_SHARED((NPAD, H), jnp.float32)]
      + [pltpu.SemaphoreType.DMA] * (2 * NBUF + 4),
)
def _sc_segment_sum(h_hbm, src_hbm, dst_hbm, out_hbm,
                    srcb, dstb, *rest):
    rows = rest[:NBUF]
    acc_sh = rest[NBUF]
    gsem = rest[NBUF + 1:2 * NBUF + 1]
    ssem = rest[2 * NBUF + 1:3 * NBUF + 1]
    isem = rest[3 * NBUF + 1:3 * NBUF + 3]
    jsem = rest[3 * NBUF + 3:3 * NBUF + 5]
    c = lax.axis_index("c")
    s = lax.axis_index("s")
    zero16 = jnp.full((LANES,), 0.0, jnp.float32)

    def zero_body(e, carry):
        for g in range(H // LANES):
            rows[0][e, pl.ds(g * LANES, LANES)] = zero16
        return carry

    lax.fori_loop(0, GCH, zero_body, 0)
    for q in range(RPT // GCH):
        pltpu.sync_copy(rows[0], acc_sh.at[pl.ds(s * RPT + q * GCH, GCH)])
    for p in range(2):
        pltpu.async_copy(src_hbm.at[c, s, p], srcb.at[p], isem[p])
        pltpu.async_copy(dst_hbm.at[c, s, p], dstb.at[p], jsem[p])
    plsc.subcore_barrier()

    def body(jj, carry):
        for p in range(2):
            g = jj * 2 + p
            pltpu.make_async_copy(src_hbm.at[c, s, g], srcb.at[p],
                                  isem[p]).wait()
            pltpu.make_async_copy(dst_hbm.at[c, s, g], dstb.at[p],
                                  jsem[p]).wait()
            gd = [pltpu.async_copy(h_hbm.at[srcb.at[p, b]], rows[b], gsem[b])
                  for b in range(NBUF)]
            sd = []
            for b in range(NBUF):
                gd[b].wait()
                sd.append(pltpu.async_copy(rows[b], acc_sh.at[dstb.at[p, b]],
                                           ssem[b], add=True))
            for d in sd:
                d.wait()
            nxt = g + 2

            @pl.when(nxt < NGRPS)
            def _():
                pltpu.async_copy(src_hbm.at[c, s, nxt], srcb.at[p], isem[p])
                pltpu.async_copy(dst_hbm.at[c, s, nxt], dstb.at[p], jsem[p])
        return carry

    lax.fori_loop(0, NGRPS // 2, body, 0)
    plsc.subcore_barrier()
    pltpu.sync_copy(acc_sh.at[pl.ds(s * RPT, RPT)],
                    out_hbm.at[c, pl.ds(s * RPT, RPT)])


@functools.partial(
    pl.kernel,
    out_type=jax.ShapeDtypeStruct((R, NPAD, H), jnp.float32),
    mesh=_sc_mesh,
    scratch_types=[
        pltpu.VMEM((NCHUNK, CH), jnp.int32),
        pltpu.VMEM((NCHUNK * CH,), jnp.float32),
        pltpu.VMEM((CH, H), jnp.float32),
        pltpu.VMEM((H,), jnp.float32),
        pltpu.VMEM((H,), jnp.float32),
        pltpu.VMEM_SHARED((NPAD, H), jnp.float32),
        pltpu.SemaphoreType.DMA,
    ],
)
def _sc_enh(base_hbm, wvec_hbm, w_hbm, dst_hbm, enh_hbm,
            dst_v, w_v, rows_v, base_v, wvec_v, acc_sh, sem):
    c = lax.axis_index("c")
    s = lax.axis_index("s")
    zero16 = jnp.full((LANES,), 0.0, jnp.float32)

    def zero_body(e, carry):
        for g in range(H // LANES):
            rows_v[e, pl.ds(g * LANES, LANES)] = zero16
        return carry

    lax.fori_loop(0, CH, zero_body, 0)
    for q in range(RPT // CH):
        pltpu.sync_copy(rows_v, acc_sh.at[pl.ds(s * RPT + q * CH, CH)])
    pltpu.sync_copy(base_hbm.at[c], base_v)
    pltpu.sync_copy(wvec_hbm, wvec_v)
    pltpu.sync_copy(dst_hbm.at[c, s], dst_v)
    pltpu.sync_copy(w_hbm.at[c, s], w_v)

    base_g = [base_v[pl.ds(g * LANES, LANES)] for g in range(H // LANES)]
    wvec_g = [wvec_v[pl.ds(g * LANES, LANES)] for g in range(H // LANES)]
    plsc.subcore_barrier()

    def chunk_body(j, carry):
        jbase = j * CH

        def grp_body(eb, carry2):
            wv16 = w_v[pl.ds(jbase + eb * LANES, LANES)]
            e0 = eb * LANES
            for k in range(LANES):
                wk = jnp.full((LANES,), wv16[k])
                for g in range(H // LANES):
                    rows_v[e0 + k, pl.ds(g * LANES, LANES)] = jnp.maximum(
                        base_g[g] + wk * wvec_g[g], 0.0)
            return carry2

        lax.fori_loop(0, CH // LANES, grp_body, 0)
        pltpu.sync_copy(rows_v, acc_sh.at[dst_v.at[j]], add=True)
        return carry

    lax.fori_loop(0, NCHUNK, chunk_body, 0)
    plsc.subcore_barrier()
    pltpu.sync_copy(acc_sh.at[pl.ds(s * RPT, RPT)],
                    enh_hbm.at[c, pl.ds(s * RPT, RPT)])


@functools.partial(
    pl.kernel,
    out_type=jax.ShapeDtypeStruct((R, NPAD, H), jnp.float32),
    mesh=_sc_mesh,
    scratch_types=[
        pltpu.VMEM((NCHUNK, CH), jnp.int32),
        pltpu.VMEM((CH, H), jnp.float32),
        pltpu.VMEM_SHARED((NPAD, H), jnp.float32),
        pltpu.SemaphoreType.DMA,
    ],
)
def _sc_counts(dst_hbm, cnt_hbm, dst_v, ones_v, acc_sh, sem):
    c = lax.axis_index("c")
    s = lax.axis_index("s")
    zero16 = jnp.full((LANES,), 0.0, jnp.float32)
    one0 = jnp.where(lax.iota(jnp.int32, LANES) == 0,
                     jnp.full((LANES,), 1.0, jnp.float32),
                     zero16)

    def zero_body(e, carry):
        for g in range(H // LANES):
            ones_v[e, pl.ds(g * LANES, LANES)] = zero16
        return carry

    lax.fori_loop(0, CH, zero_body, 0)
    for q in range(RPT // CH):
        pltpu.sync_copy(ones_v, acc_sh.at[pl.ds(s * RPT + q * CH, CH)])

    def ones_body(e, carry):
        ones_v[e, pl.ds(0, LANES)] = one0
        return carry

    lax.fori_loop(0, CH, ones_body, 0)
    pltpu.sync_copy(dst_hbm.at[c, s], dst_v)
    plsc.subcore_barrier()

    def cnt_body(j, carry):
        pltpu.sync_copy(ones_v, acc_sh.at[dst_v.at[j]], add=True)
        return carry

    lax.fori_loop(0, NCHUNK, cnt_body, 0)
    plsc.subcore_barrier()
    pltpu.sync_copy(acc_sh.at[pl.ds(s * RPT, RPT)],
                    cnt_hbm.at[c, pl.ds(s * RPT, RPT)])


MGCH = 32
MNBUF = 2
MNGRP = EPT // (MNBUF * MGCH)
NHALF = 5120
ACCR = 5128
ARPT = 320
SCRAPM = 5120


def _make_halfseg(k):
    @functools.partial(
        pl.kernel,
        out_type=jax.ShapeDtypeStruct((R, ACCR, H), jnp.float32),
        mesh=_sc_mesh,
        scratch_types=[
            pltpu.VMEM((2, MNBUF, MGCH), jnp.int32),
            pltpu.VMEM((2, MNBUF, MGCH), jnp.int32),
            pltpu.VMEM((MNBUF, MGCH), jnp.int32),
            pltpu.VMEM((MGCH,), jnp.int32),
            pltpu.VMEM((LANES,), jnp.int32),
        ] + [pltpu.VMEM((MGCH, H), jnp.float32)] * MNBUF
          + [pltpu.VMEM_SHARED((N, H), jnp.float32)]
          + [pltpu.VMEM_SHARED((ACCR, H), jnp.float32)]
          + [pltpu.SemaphoreType.DMA] * (2 * MNBUF + 4),
    )
    def halfseg(h_hbm, src_hbm, dst_hbm, cnt_hbm, out_hbm,
                srcb, dstb, dclamp, scrap_v, cnt_v, *rest):
        rows = rest[:MNBUF]
        mir_sh = rest[MNBUF]
        acc_sh = rest[MNBUF + 1]
        gsem = rest[MNBUF + 2:2 * MNBUF + 2]
        ssem = rest[2 * MNBUF + 2:2 * MNBUF + 4]
        isem = rest[2 * MNBUF + 4:2 * MNBUF + 6]
        jsem = rest[2 * MNBUF + 6:2 * MNBUF + 8]
        c = lax.axis_index("c")
        s = lax.axis_index("s")
        zero16 = jnp.full((LANES,), 0.0, jnp.float32)
        scrap16 = jnp.full((LANES,), SCRAPM, jnp.int32)
        half16 = jnp.full((LANES,), NHALF, jnp.int32)

        def zero_body(e, carry):
            for g in range(H // LANES):
                rows[0][e, pl.ds(g * LANES, LANES)] = zero16
            return carry

        lax.fori_loop(0, MGCH, zero_body, 0)
        for kk in range(MGCH // LANES):
            scrap_v[pl.ds(kk * LANES, LANES)] = scrap16
        for q in range(ARPT // MGCH):
            pltpu.sync_copy(rows[0],
                            acc_sh.at[pl.ds(s * ARPT + q * MGCH, MGCH)])

        @pl.when(s == NS - 1)
        def _():
            pltpu.sync_copy(rows[0].at[pl.ds(0, ACCR - NHALF)],
                            acc_sh.at[pl.ds(NHALF, ACCR - NHALF)])
        @pl.when(s < NS - 1)
        def _():
            pltpu.sync_copy(h_hbm.at[pl.ds(s * RPT, RPT)],
                            mir_sh.at[pl.ds(s * RPT, RPT)])

        @pl.when(s == NS - 1)
        def _():
            pltpu.sync_copy(h_hbm.at[pl.ds((NS - 1) * RPT, N - (NS - 1) * RPT)],
                            mir_sh.at[pl.ds((NS - 1) * RPT, N - (NS - 1) * RPT)])

        pltpu.sync_copy(cnt_hbm.at[c], cnt_v)
        cnt0 = cnt_v[pl.ds(0, LANES)][0]
        l1 = jnp.clip(cnt0 - s * EPT, 0, EPT)
        grpsz = MNBUF * MGCH
        if k == 0:
            glo = jnp.int32(0)
            ghi = (l1 + grpsz - 1) // grpsz
        else:
            glo = l1 // grpsz
            ghi = jnp.int32(MNGRP)
        n_pairs = (ghi - glo + 1) // 2
        for p in range(2):
            @pl.when(glo + p < ghi)
            def _(p=p):
                pltpu.async_copy(src_hbm.at[c, s, glo + p], srcb.at[p],
                                 isem[p])
                pltpu.async_copy(dst_hbm.at[c, s, glo + p], dstb.at[p],
                                 jsem[p])
        plsc.subcore_barrier()
        pltpu.async_copy(rows[0], acc_sh.at[scrap_v], ssem[0], add=True)
        pltpu.async_copy(rows[1], acc_sh.at[scrap_v], ssem[1], add=True)

        def body(jj, carry):
            for p in range(2):
                g = glo + jj * 2 + p

                @pl.when(g < ghi)
                def _(p=p, g=g):
                    pltpu.make_async_copy(src_hbm.at[c, s, g], srcb.at[p],
                                          isem[p]).wait()
                    pltpu.make_async_copy(dst_hbm.at[c, s, g], dstb.at[p],
                                          jsem[p]).wait()
                    gd = [pltpu.async_copy(mir_sh.at[srcb.at[p, b]], rows[b],
                                           gsem[b]) for b in range(MNBUF)]
                    for b in range(MNBUF):
                        gd[b].wait()
                        pltpu.make_async_copy(rows[b], acc_sh.at[scrap_v],
                                              ssem[b]).wait()
                        for v in range(MGCH // LANES):
                            dv = dstb[p, b, pl.ds(v * LANES, LANES)]
                            if k == 0:
                                d = jnp.where(dv < half16, dv, scrap16)
                            else:
                                d = jnp.where(dv >= half16, dv - half16,
                                              scrap16)
                            dclamp[b, pl.ds(v * LANES, LANES)] = d
                        pltpu.async_copy(rows[b], acc_sh.at[dclamp.at[b]],
                                         ssem[b], add=True)
                    nxt = g + 2

                    @pl.when(nxt < ghi)
                    def _():
                        pltpu.async_copy(src_hbm.at[c, s, nxt], srcb.at[p],
                                         isem[p])
                        pltpu.async_copy(dst_hbm.at[c, s, nxt], dstb.at[p],
                                         jsem[p])
            return carry

        lax.fori_loop(0, n_pairs, body, 0)
        for b in range(MNBUF):
            pltpu.make_async_copy(rows[b], acc_sh.at[scrap_v], ssem[b]).wait()
        plsc.subcore_barrier()
        pltpu.sync_copy(acc_sh.at[pl.ds(s * ARPT, ARPT)],
                        out_hbm.at[c, pl.ds(s * ARPT, ARPT)])

        @pl.when(s == NS - 1)
        def _():
            pltpu.sync_copy(acc_sh.at[pl.ds(NHALF, ACCR - NHALF)],
                            out_hbm.at[c, pl.ds(NHALF, ACCR - NHALF)])

    return halfseg


_halfseg0 = _make_halfseg(0)
_halfseg1 = _make_halfseg(1)


def _ln(y, g, b):
    mu = jnp.mean(y, axis=-1, keepdims=True)
    var = jnp.mean((y - mu) ** 2, axis=-1, keepdims=True)
    return (y - mu) * lax.rsqrt(var + 1e-5) * g + b


def _proj_body(x_ref, w_ref, b_ref, g_ref, bb_ref, out_ref):
    y = jnp.dot(x_ref[...], w_ref[...], preferred_element_type=jnp.float32)
    y = y + b_ref[...]
    out_ref[...] = jnp.maximum(_ln(y, g_ref[...], bb_ref[...]), 0.0)


def _base_body(emb_ref, w_ref, b_ref, out_ref):
    out_ref[...] = jnp.dot(emb_ref[...], w_ref[...],
                           preferred_element_type=jnp.float32) + b_ref[...]


def _wcomb_body(comp_ref, basis_ref, out_ref):
    out_ref[0] = jnp.dot(comp_ref[0], basis_ref[0],
                         preferred_element_type=jnp.float32)


def _layer_body(h_ref, slo_ref, shi_ref, enh_ref, cnt_ref, w_ref, root_ref,
                bias_ref, g_ref, b_ref, out_ref):
    c0 = cnt_ref[0]
    c1 = cnt_ref[1]
    deg = jnp.maximum(c0 + c1, 1.0)
    es = 0.1 * (enh_ref[0] + enh_ref[1]) / deg
    use_lo = pl.program_id(0) < NHALF // _ROWB
    s0 = jnp.where(use_lo, slo_ref[0], shi_ref[0])
    s1 = jnp.where(use_lo, slo_ref[1], shi_ref[1])
    m0 = s0 / jnp.maximum(c0, 1.0)
    m1 = s1 / jnp.maximum(c1, 1.0)
    h = h_ref[...]
    agg = (jnp.dot(m0, w_ref[0, 0], preferred_element_type=jnp.float32)
           + jnp.dot(m1, w_ref[0, 1], preferred_element_type=jnp.float32)
           + jnp.dot(h, root_ref[0], preferred_element_type=jnp.float32)
           + bias_ref[0] + es)
    out_ref[...] = jnp.maximum(_ln(agg, g_ref[0], b_ref[0]), 0.0) + h


_ROWB = 1024
_NBLK = (N + _ROWB - 1) // _ROWB


def _tc_proj(x, w, b, g, bb):
    return pl.pallas_call(
        _proj_body,
        grid=(_NBLK,),
        in_specs=[
            pl.BlockSpec((_ROWB, D_IN), lambda i: (i, 0)),
            pl.BlockSpec((D_IN, H), lambda i: (0, 0)),
            pl.BlockSpec((1, H), lambda i: (0, 0)),
            pl.BlockSpec((1, H), lambda i: (0, 0)),
            pl.BlockSpec((1, H), lambda i: (0, 0)),
        ],
        out_specs=pl.BlockSpec((_ROWB, H), lambda i: (i, 0)),
        out_shape=jax.ShapeDtypeStruct((N, H), jnp.float32),
    )(x, w, b, g, bb)


def _tc_base(emb, w16, b):
    return pl.pallas_call(
        _base_body,
        out_shape=jax.ShapeDtypeStruct((R, H), jnp.float32),
    )(emb, w16, b)


def _tc_wcomb(comp, basis_r):
    return pl.pallas_call(
        _wcomb_body,
        grid=(L,),
        in_specs=[
            pl.BlockSpec((1, R, B), lambda l: (l, 0, 0)),
            pl.BlockSpec((1, B, H * H), lambda l: (l, 0, 0)),
        ],
        out_specs=pl.BlockSpec((1, R, H * H), lambda l: (l, 0, 0)),
        out_shape=jax.ShapeDtypeStruct((L, R, H * H), jnp.float32),
    )(comp, basis_r)


def _tc_layer(l, h, s_lo, s_hi, enh, cnt, w3, root, bias, ng, nb):
    nlo = NHALF // _ROWB
    return pl.pallas_call(
        functools.partial(_layer_body),
        grid=(_NBLK,),
        in_specs=[
            pl.BlockSpec((_ROWB, H), lambda i: (i, 0)),
            pl.BlockSpec((R, _ROWB, H),
                         lambda i, _n=nlo: (0, jnp.minimum(i, _n - 1), 0)),
            pl.BlockSpec((R, _ROWB, H),
                         lambda i, _n=nlo: (0, jnp.maximum(i - _n, 0), 0)),
            pl.BlockSpec((R, _ROWB, H), lambda i: (0, i, 0)),
            pl.BlockSpec((R, _ROWB, 1), lambda i: (0, i, 0)),
            pl.BlockSpec((1, R, H, H), lambda i, _l=l: (_l, 0, 0, 0)),
            pl.BlockSpec((1, H, H), lambda i, _l=l: (_l, 0, 0)),
            pl.BlockSpec((1, 1, H), lambda i, _l=l: (_l, 0, 0)),
            pl.BlockSpec((1, 1, H), lambda i, _l=l: (_l, 0, 0)),
            pl.BlockSpec((1, 1, H), lambda i, _l=l: (_l, 0, 0)),
        ],
        out_specs=pl.BlockSpec((_ROWB, H), lambda i: (i, 0)),
        out_shape=jax.ShapeDtypeStruct((N, H), jnp.float32),
    )(h, s_lo, s_hi, enh, cnt, w3, root, bias, ng, nb)


def _partition_rel(src_r, dst_r):
    src_r = src_r.astype(jnp.int32)
    dst_r = dst_r.astype(jnp.int32)
    half = (dst_r >= NHALF).astype(jnp.int32)
    cnt0 = E - half.sum()
    packed = (half << 28) | (src_r << 14) | dst_r
    packed = jax.lax.sort(packed, is_stable=False)
    dp = packed & jnp.int32(16383)
    sp = (packed >> 14) & jnp.int32(16383)
    pad = EPAD - E
    sp = jnp.concatenate([sp, jnp.zeros((pad,), jnp.int32)])
    dp = jnp.concatenate([dp, jnp.full((pad,), 2 * NHALF, jnp.int32)])
    return sp, dp, cnt0


def _prep_rel(idx_row, pad_val, dtype):
    pad = EPAD - E
    arr = jnp.concatenate(
        [idx_row.astype(dtype), jnp.full((pad,), pad_val, dtype)])
    return arr.reshape(NS, NCHUNK, CH)


def kernel(x, edge_index_r0, edge_index_r1, edge_attr_r0, edge_attr_r1,
           proj_W, proj_b, proj_ln_g, proj_ln_b, edge_emb, emlp_W, emlp_b,
           conv_comp, conv_basis, conv_root, conv_bias, norm_g, norm_b):
    src = jnp.stack([_prep_rel(edge_index_r0[0], 0, jnp.int32),
                     _prep_rel(edge_index_r1[0], 0, jnp.int32)])
    dst = jnp.stack([_prep_rel(edge_index_r0[1], N, jnp.int32),
                     _prep_rel(edge_index_r1[1], N, jnp.int32)])
    sp0, dp0, cnt0_0 = _partition_rel(edge_index_r0[0], edge_index_r0[1])
    sp1, dp1, cnt0_1 = _partition_rel(edge_index_r1[0], edge_index_r1[1])
    src_m = jnp.stack([sp0, sp1]).reshape(R, NS, MNGRP, MNBUF, MGCH)
    dst_m = jnp.stack([dp0, dp1]).reshape(R, NS, MNGRP, MNBUF, MGCH)
    cnt_arr = jnp.broadcast_to(
        jnp.stack([cnt0_0, cnt0_1]).astype(jnp.int32)[:, None], (R, LANES))
    cnt_arr = cnt_arr + jnp.zeros((R, LANES), jnp.int32)
    wgt = jnp.stack([_prep_rel(edge_attr_r0[:, 1], 0.0, jnp.float32),
                     _prep_rel(edge_attr_r1[:, 1], 0.0, jnp.float32)]
                    ).reshape(R, NS, NCHUNK * CH)

    h = _tc_proj(x, proj_W, proj_b.reshape(1, H), proj_ln_g.reshape(1, H),
                 proj_ln_b.reshape(1, H))
    base = _tc_base(edge_emb, emlp_W[:EDIM], emlp_b.reshape(1, H))
    w3 = _tc_wcomb(conv_comp, conv_basis.reshape(L, B, H * H)
                   ).reshape(L, R, H, H)

    enh = _sc_enh(base, emlp_W[EDIM], wgt, dst)
    cnt_full = _sc_counts(dst)
    cnt = cnt_full[:, :, 0:1]

    for l in range(L):
        s_lo = _halfseg0(h, src_m, dst_m, cnt_arr)
        s_hi = _halfseg1(h, src_m, dst_m, cnt_arr)
        h = _tc_layer(l, h, s_lo, s_hi, enh, cnt, w3, conv_root,
                      conv_bias.reshape(L, 1, H), norm_g.reshape(L, 1, H),
                      norm_b.reshape(L, 1, H))
    return h


# speedup vs baseline: 3.3090x; 1.1865x over previous
import functools

import jax
import jax.numpy as jnp
from jax import lax
from jax.experimental import pallas as pl
from jax.experimental.pallas import tpu as pltpu, tpu_sc as plsc

N = 10000
E = 160000
D_IN = 128
H = 128
R = 2
B = 8
EDIM = 16
L = 3

NC, NS, LANES = 2, 16, 16
CH = 128
NCHUNK = 80
EPT = NCHUNK * CH
EPAD = NS * EPT
NPAD = 10240
RPT = NPAD // NS
HE = H + LANES

_sc_mesh = plsc.VectorSubcoreMesh(core_axis_name="c", subcore_axis_name="s")


GCH = 64
NBUF = 4
NGRPS = EPT // (NBUF * GCH)


@functools.partial(
    pl.kernel,
    out_type=jax.ShapeDtypeStruct((R, NPAD, H), jnp.float32),
    mesh=_sc_mesh,
    scratch_types=[
        pltpu.VMEM((2, NBUF, GCH), jnp.int32),
        pltpu.VMEM((2, NBUF, GCH), jnp.int32),
    ] + [pltpu.VMEM((GCH, H), jnp.float32)] * NBUF
      + [pltpu.VMEM_SHARED((NPAD, H), jnp.float32)]
      + [pltpu.SemaphoreType.DMA] * (2 * NBUF + 4),
)
def _sc_segment_sum(h_hbm, src_hbm, dst_hbm, out_hbm,
                    srcb, dstb, *rest):
    rows = rest[:NBUF]
    acc_sh = rest[NBUF]
    gsem = rest[NBUF + 1:2 * NBUF + 1]
    ssem = rest[2 * NBUF + 1:3 * NBUF + 1]
    isem = rest[3 * NBUF + 1:3 * NBUF + 3]
    jsem = rest[3 * NBUF + 3:3 * NBUF + 5]
    c = lax.axis_index("c")
    s = lax.axis_index("s")
    zero16 = jnp.full((LANES,), 0.0, jnp.float32)

    def zero_body(e, carry):
        for g in range(H // LANES):
            rows[0][e, pl.ds(g * LANES, LANES)] = zero16
        return carry

    lax.fori_loop(0, GCH, zero_body, 0)
    for q in range(RPT // GCH):
        pltpu.sync_copy(rows[0], acc_sh.at[pl.ds(s * RPT + q * GCH, GCH)])
    for p in range(2):
        pltpu.async_copy(src_hbm.at[c, s, p], srcb.at[p], isem[p])
        pltpu.async_copy(dst_hbm.at[c, s, p], dstb.at[p], jsem[p])
    plsc.subcore_barrier()

    def body(jj, carry):
        for p in range(2):
            g = jj * 2 + p
            pltpu.make_async_copy(src_hbm.at[c, s, g], srcb.at[p],
                                  isem[p]).wait()
            pltpu.make_async_copy(dst_hbm.at[c, s, g], dstb.at[p],
                                  jsem[p]).wait()
            gd = [pltpu.async_copy(h_hbm.at[srcb.at[p, b]], rows[b], gsem[b])
                  for b in range(NBUF)]
            sd = []
            for b in range(NBUF):
                gd[b].wait()
                sd.append(pltpu.async_copy(rows[b], acc_sh.at[dstb.at[p, b]],
                                           ssem[b], add=True))
            for d in sd:
                d.wait()
            nxt = g + 2

            @pl.when(nxt < NGRPS)
            def _():
                pltpu.async_copy(src_hbm.at[c, s, nxt], srcb.at[p], isem[p])
                pltpu.async_copy(dst_hbm.at[c, s, nxt], dstb.at[p], jsem[p])
        return carry

    lax.fori_loop(0, NGRPS // 2, body, 0)
    plsc.subcore_barrier()
    pltpu.sync_copy(acc_sh.at[pl.ds(s * RPT, RPT)],
                    out_hbm.at[c, pl.ds(s * RPT, RPT)])


@functools.partial(
    pl.kernel,
    out_type=jax.ShapeDtypeStruct((R, NPAD, H), jnp.float32),
    mesh=_sc_mesh,
    scratch_types=[
        pltpu.VMEM((NCHUNK, CH), jnp.int32),
        pltpu.VMEM((NCHUNK * CH,), jnp.float32),
        pltpu.VMEM((CH, H), jnp.float32),
        pltpu.VMEM((H,), jnp.float32),
        pltpu.VMEM((H,), jnp.float32),
        pltpu.VMEM_SHARED((NPAD, H), jnp.float32),
        pltpu.SemaphoreType.DMA,
    ],
)
def _sc_enh(base_hbm, wvec_hbm, w_hbm, dst_hbm, enh_hbm,
            dst_v, w_v, rows_v, base_v, wvec_v, acc_sh, sem):
    c = lax.axis_index("c")
    s = lax.axis_index("s")
    zero16 = jnp.full((LANES,), 0.0, jnp.float32)

    def zero_body(e, carry):
        for g in range(H // LANES):
            rows_v[e, pl.ds(g * LANES, LANES)] = zero16
        return carry

    lax.fori_loop(0, CH, zero_body, 0)
    for q in range(RPT // CH):
        pltpu.sync_copy(rows_v, acc_sh.at[pl.ds(s * RPT + q * CH, CH)])
    pltpu.sync_copy(base_hbm.at[c], base_v)
    pltpu.sync_copy(wvec_hbm, wvec_v)
    pltpu.sync_copy(dst_hbm.at[c, s], dst_v)
    pltpu.sync_copy(w_hbm.at[c, s], w_v)

    base_g = [base_v[pl.ds(g * LANES, LANES)] for g in range(H // LANES)]
    wvec_g = [wvec_v[pl.ds(g * LANES, LANES)] for g in range(H // LANES)]
    plsc.subcore_barrier()

    def chunk_body(j, carry):
        jbase = j * CH

        def grp_body(eb, carry2):
            wv16 = w_v[pl.ds(jbase + eb * LANES, LANES)]
            e0 = eb * LANES
            for k in range(LANES):
                wk = jnp.full((LANES,), wv16[k])
                for g in range(H // LANES):
                    rows_v[e0 + k, pl.ds(g * LANES, LANES)] = jnp.maximum(
                        base_g[g] + wk * wvec_g[g], 0.0)
            return carry2

        lax.fori_loop(0, CH // LANES, grp_body, 0)
        pltpu.sync_copy(rows_v, acc_sh.at[dst_v.at[j]], add=True)
        return carry

    lax.fori_loop(0, NCHUNK, chunk_body, 0)
    plsc.subcore_barrier()
    pltpu.sync_copy(acc_sh.at[pl.ds(s * RPT, RPT)],
                    enh_hbm.at[c, pl.ds(s * RPT, RPT)])


@functools.partial(
    pl.kernel,
    out_type=jax.ShapeDtypeStruct((R, NPAD, H), jnp.float32),
    mesh=_sc_mesh,
    scratch_types=[
        pltpu.VMEM((NCHUNK, CH), jnp.int32),
        pltpu.VMEM((CH, H), jnp.float32),
        pltpu.VMEM_SHARED((NPAD, H), jnp.float32),
        pltpu.SemaphoreType.DMA,
    ],
)
def _sc_counts(dst_hbm, cnt_hbm, dst_v, ones_v, acc_sh, sem):
    c = lax.axis_index("c")
    s = lax.axis_index("s")
    zero16 = jnp.full((LANES,), 0.0, jnp.float32)
    one0 = jnp.where(lax.iota(jnp.int32, LANES) == 0,
                     jnp.full((LANES,), 1.0, jnp.float32),
                     zero16)

    def zero_body(e, carry):
        for g in range(H // LANES):
            ones_v[e, pl.ds(g * LANES, LANES)] = zero16
        return carry

    lax.fori_loop(0, CH, zero_body, 0)
    for q in range(RPT // CH):
        pltpu.sync_copy(ones_v, acc_sh.at[pl.ds(s * RPT + q * CH, CH)])

    def ones_body(e, carry):
        ones_v[e, pl.ds(0, LANES)] = one0
        return carry

    lax.fori_loop(0, CH, ones_body, 0)
    pltpu.sync_copy(dst_hbm.at[c, s], dst_v)
    plsc.subcore_barrier()

    def cnt_body(j, carry):
        pltpu.sync_copy(ones_v, acc_sh.at[dst_v.at[j]], add=True)
        return carry

    lax.fori_loop(0, NCHUNK, cnt_body, 0)
    plsc.subcore_barrier()
    pltpu.sync_copy(acc_sh.at[pl.ds(s * RPT, RPT)],
                    cnt_hbm.at[c, pl.ds(s * RPT, RPT)])


MGCH = 32
MNBUF = 2
MNGRP = EPT // (MNBUF * MGCH)
NHALF = 5120
ACCR = 5128
ARPT = 320
SCRAPM = 5120


@functools.partial(
    pl.kernel,
    out_type=(jax.ShapeDtypeStruct((R, ACCR, H), jnp.float32),
              jax.ShapeDtypeStruct((R, ACCR, H), jnp.float32)),
    mesh=_sc_mesh,
    scratch_types=[
        pltpu.VMEM((2, MNBUF, MGCH), jnp.int32),
        pltpu.VMEM((2, MNBUF, MGCH), jnp.int32),
        pltpu.VMEM((MNBUF, MGCH), jnp.int32),
        pltpu.VMEM((MGCH,), jnp.int32),
        pltpu.VMEM((LANES,), jnp.int32),
    ] + [pltpu.VMEM((MGCH, H), jnp.float32)] * MNBUF
      + [pltpu.VMEM_SHARED((N, H), jnp.float32)]
      + [pltpu.VMEM_SHARED((ACCR, H), jnp.float32)]
      + [pltpu.SemaphoreType.DMA] * (2 * MNBUF + 4),
)
def _sc_seg_both(h_hbm, src_hbm, dst_hbm, cnt_hbm, lo_hbm, hi_hbm,
                 srcb, dstb, dclamp, scrap_v, cnt_v, *rest):
    rows = rest[:MNBUF]
    mir_sh = rest[MNBUF]
    acc_sh = rest[MNBUF + 1]
    gsem = rest[MNBUF + 2:2 * MNBUF + 2]
    ssem = rest[2 * MNBUF + 2:2 * MNBUF + 4]
    isem = rest[2 * MNBUF + 4:2 * MNBUF + 6]
    jsem = rest[2 * MNBUF + 6:2 * MNBUF + 8]
    outs = (lo_hbm, hi_hbm)
    c = lax.axis_index("c")
    s = lax.axis_index("s")
    zero16 = jnp.full((LANES,), 0.0, jnp.float32)
    scrap16 = jnp.full((LANES,), SCRAPM, jnp.int32)
    half16 = jnp.full((LANES,), NHALF, jnp.int32)

    def zero_body(e, carry):
        for g in range(H // LANES):
            rows[0][e, pl.ds(g * LANES, LANES)] = zero16
        return carry

    for kk in range(MGCH // LANES):
        scrap_v[pl.ds(kk * LANES, LANES)] = scrap16
    @pl.when(s < NS - 1)
    def _():
        pltpu.sync_copy(h_hbm.at[pl.ds(s * RPT, RPT)],
                        mir_sh.at[pl.ds(s * RPT, RPT)])

    @pl.when(s == NS - 1)
    def _():
        pltpu.sync_copy(h_hbm.at[pl.ds((NS - 1) * RPT, N - (NS - 1) * RPT)],
                        mir_sh.at[pl.ds((NS - 1) * RPT, N - (NS - 1) * RPT)])

    pltpu.sync_copy(cnt_hbm.at[c], cnt_v)
    cnt0 = cnt_v[pl.ds(0, LANES)][0]
    l1 = jnp.clip(cnt0 - s * EPT, 0, EPT)
    grpsz = MNBUF * MGCH

    for k in range(2):
        lax.fori_loop(0, MGCH, zero_body, 0)
        for q in range(ARPT // MGCH):
            pltpu.sync_copy(rows[0],
                            acc_sh.at[pl.ds(s * ARPT + q * MGCH, MGCH)])

        @pl.when(s == NS - 1)
        def _():
            pltpu.sync_copy(rows[0].at[pl.ds(0, ACCR - NHALF)],
                            acc_sh.at[pl.ds(NHALF, ACCR - NHALF)])

        if k == 0:
            glo = jnp.int32(0)
            ghi = (l1 + grpsz - 1) // grpsz
        else:
            glo = l1 // grpsz
            ghi = jnp.int32(MNGRP)
        n_pairs = (ghi - glo + 1) // 2
        for p in range(2):
            @pl.when(glo + p < ghi)
            def _(p=p, glo=glo, ghi=ghi):
                pltpu.async_copy(src_hbm.at[c, s, glo + p], srcb.at[p],
                                 isem[p])
                pltpu.async_copy(dst_hbm.at[c, s, glo + p], dstb.at[p],
                                 jsem[p])
        plsc.subcore_barrier()
        pltpu.async_copy(rows[0], acc_sh.at[scrap_v], ssem[0], add=True)
        pltpu.async_copy(rows[1], acc_sh.at[scrap_v], ssem[1], add=True)

        def body(jj, carry, k=k, glo=glo, ghi=ghi):
            for p in range(2):
                g = glo + jj * 2 + p

                @pl.when(g < ghi)
                def _(p=p, g=g):
                    pltpu.make_async_copy(src_hbm.at[c, s, g], srcb.at[p],
                                          isem[p]).wait()
                    pltpu.make_async_copy(dst_hbm.at[c, s, g], dstb.at[p],
                                          jsem[p]).wait()
                    gd = [pltpu.async_copy(mir_sh.at[srcb.at[p, b]], rows[b],
                                           gsem[b]) for b in range(MNBUF)]
                    for b in range(MNBUF):
                        gd[b].wait()
                        pltpu.make_async_copy(rows[b], acc_sh.at[scrap_v],
                                              ssem[b]).wait()
                        for v in range(MGCH // LANES):
                            dv = dstb[p, b, pl.ds(v * LANES, LANES)]
                            if k == 0:
                                d = jnp.where(dv < half16, dv, scrap16)
                            else:
                                d = jnp.where(dv >= half16, dv - half16,
                                              scrap16)
                            dclamp[b, pl.ds(v * LANES, LANES)] = d
                        pltpu.async_copy(rows[b], acc_sh.at[dclamp.at[b]],
                                         ssem[b], add=True)
                    nxt = g + 2

                    @pl.when(nxt < ghi)
                    def _():
                        pltpu.async_copy(src_hbm.at[c, s, nxt], srcb.at[p],
                                         isem[p])
                        pltpu.async_copy(dst_hbm.at[c, s, nxt], dstb.at[p],
                                         jsem[p])
            return carry

        lax.fori_loop(0, n_pairs, body, 0)
        for b in range(MNBUF):
            pltpu.make_async_copy(rows[b], acc_sh.at[scrap_v],
                                  ssem[b]).wait()
        plsc.subcore_barrier()
        pltpu.sync_copy(acc_sh.at[pl.ds(s * ARPT, ARPT)],
                        outs[k].at[c, pl.ds(s * ARPT, ARPT)])

        @pl.when(s == NS - 1)
        def _(k=k):
            pltpu.sync_copy(acc_sh.at[pl.ds(NHALF, ACCR - NHALF)],
                            outs[k].at[c, pl.ds(NHALF, ACCR - NHALF)])


def _ln(y, g, b):
    mu = jnp.mean(y, axis=-1, keepdims=True)
    var = jnp.mean((y - mu) ** 2, axis=-1, keepdims=True)
    return (y - mu) * lax.rsqrt(var + 1e-5) * g + b


def _proj_body(x_ref, w_ref, b_ref, g_ref, bb_ref, out_ref):
    y = jnp.dot(x_ref[...], w_ref[...], preferred_element_type=jnp.float32)
    y = y + b_ref[...]
    out_ref[...] = jnp.maximum(_ln(y, g_ref[...], bb_ref[...]), 0.0)


def _base_body(emb_ref, w_ref, b_ref, out_ref):
    out_ref[...] = jnp.dot(emb_ref[...], w_ref[...],
                           preferred_element_type=jnp.float32) + b_ref[...]


def _wcomb_body(comp_ref, basis_ref, out_ref):
    out_ref[0] = jnp.dot(comp_ref[0], basis_ref[0],
                         preferred_element_type=jnp.float32)


def _layer_body(h_ref, slo_ref, shi_ref, enh_ref, cnt_ref, w_ref, root_ref,
                bias_ref, g_ref, b_ref, out_ref):
    c0 = cnt_ref[0]
    c1 = cnt_ref[1]
    deg = jnp.maximum(c0 + c1, 1.0)
    es = 0.1 * (enh_ref[0] + enh_ref[1]) / deg
    use_lo = pl.program_id(0) < NHALF // _ROWB
    s0 = jnp.where(use_lo, slo_ref[0], shi_ref[0])
    s1 = jnp.where(use_lo, slo_ref[1], shi_ref[1])
    m0 = s0 / jnp.maximum(c0, 1.0)
    m1 = s1 / jnp.maximum(c1, 1.0)
    h = h_ref[...]
    agg = (jnp.dot(m0, w_ref[0, 0], preferred_element_type=jnp.float32)
           + jnp.dot(m1, w_ref[0, 1], preferred_element_type=jnp.float32)
           + jnp.dot(h, root_ref[0], preferred_element_type=jnp.float32)
           + bias_ref[0] + es)
    out_ref[...] = jnp.maximum(_ln(agg, g_ref[0], b_ref[0]), 0.0) + h


_ROWB = 1024
_NBLK = (N + _ROWB - 1) // _ROWB


def _tc_proj(x, w, b, g, bb):
    return pl.pallas_call(
        _proj_body,
        grid=(_NBLK,),
        in_specs=[
            pl.BlockSpec((_ROWB, D_IN), lambda i: (i, 0)),
            pl.BlockSpec((D_IN, H), lambda i: (0, 0)),
            pl.BlockSpec((1, H), lambda i: (0, 0)),
            pl.BlockSpec((1, H), lambda i: (0, 0)),
            pl.BlockSpec((1, H), lambda i: (0, 0)),
        ],
        out_specs=pl.BlockSpec((_ROWB, H), lambda i: (i, 0)),
        out_shape=jax.ShapeDtypeStruct((N, H), jnp.float32),
    )(x, w, b, g, bb)


def _tc_base(emb, w16, b):
    return pl.pallas_call(
        _base_body,
        out_shape=jax.ShapeDtypeStruct((R, H), jnp.float32),
    )(emb, w16, b)


def _tc_wcomb(comp, basis_r):
    return pl.pallas_call(
        _wcomb_body,
        grid=(L,),
        in_specs=[
            pl.BlockSpec((1, R, B), lambda l: (l, 0, 0)),
            pl.BlockSpec((1, B, H * H), lambda l: (l, 0, 0)),
        ],
        out_specs=pl.BlockSpec((1, R, H * H), lambda l: (l, 0, 0)),
        out_shape=jax.ShapeDtypeStruct((L, R, H * H), jnp.float32),
    )(comp, basis_r)


def _tc_layer(l, h, s_lo, s_hi, enh, cnt, w3, root, bias, ng, nb):
    nlo = NHALF // _ROWB
    return pl.pallas_call(
        functools.partial(_layer_body),
        grid=(_NBLK,),
        in_specs=[
            pl.BlockSpec((_ROWB, H), lambda i: (i, 0)),
            pl.BlockSpec((R, _ROWB, H),
                         lambda i, _n=nlo: (0, jnp.minimum(i, _n - 1), 0)),
            pl.BlockSpec((R, _ROWB, H),
                         lambda i, _n=nlo: (0, jnp.maximum(i - _n, 0), 0)),
            pl.BlockSpec((R, _ROWB, H), lambda i: (0, i, 0)),
            pl.BlockSpec((R, _ROWB, 1), lambda i: (0, i, 0)),
            pl.BlockSpec((1, R, H, H), lambda i, _l=l: (_l, 0, 0, 0)),
            pl.BlockSpec((1, H, H), lambda i, _l=l: (_l, 0, 0)),
            pl.BlockSpec((1, 1, H), lambda i, _l=l: (_l, 0, 0)),
            pl.BlockSpec((1, 1, H), lambda i, _l=l: (_l, 0, 0)),
            pl.BlockSpec((1, 1, H), lambda i, _l=l: (_l, 0, 0)),
        ],
        out_specs=pl.BlockSpec((_ROWB, H), lambda i: (i, 0)),
        out_shape=jax.ShapeDtypeStruct((N, H), jnp.float32),
    )(h, s_lo, s_hi, enh, cnt, w3, root, bias, ng, nb)


def _partition_rel(src_r, dst_r):
    src_r = src_r.astype(jnp.int32)
    dst_r = dst_r.astype(jnp.int32)
    half = (dst_r >= NHALF).astype(jnp.int32)
    cnt0 = E - half.sum()
    packed = (half << 28) | (src_r << 14) | dst_r
    packed = jax.lax.sort(packed, is_stable=False)
    dp = packed & jnp.int32(16383)
    sp = (packed >> 14) & jnp.int32(16383)
    pad = EPAD - E
    sp = jnp.concatenate([sp, jnp.zeros((pad,), jnp.int32)])
    dp = jnp.concatenate([dp, jnp.full((pad,), 2 * NHALF, jnp.int32)])
    return sp, dp, cnt0


def _prep_rel(idx_row, pad_val, dtype):
    pad = EPAD - E
    arr = jnp.concatenate(
        [idx_row.astype(dtype), jnp.full((pad,), pad_val, dtype)])
    return arr.reshape(NS, NCHUNK, CH)


def kernel(x, edge_index_r0, edge_index_r1, edge_attr_r0, edge_attr_r1,
           proj_W, proj_b, proj_ln_g, proj_ln_b, edge_emb, emlp_W, emlp_b,
           conv_comp, conv_basis, conv_root, conv_bias, norm_g, norm_b):
    src = jnp.stack([_prep_rel(edge_index_r0[0], 0, jnp.int32),
                     _prep_rel(edge_index_r1[0], 0, jnp.int32)])
    dst = jnp.stack([_prep_rel(edge_index_r0[1], N, jnp.int32),
                     _prep_rel(edge_index_r1[1], N, jnp.int32)])
    sp0, dp0, cnt0_0 = _partition_rel(edge_index_r0[0], edge_index_r0[1])
    sp1, dp1, cnt0_1 = _partition_rel(edge_index_r1[0], edge_index_r1[1])
    src_m = jnp.stack([sp0, sp1]).reshape(R, NS, MNGRP, MNBUF, MGCH)
    dst_m = jnp.stack([dp0, dp1]).reshape(R, NS, MNGRP, MNBUF, MGCH)
    cnt_arr = jnp.broadcast_to(
        jnp.stack([cnt0_0, cnt0_1]).astype(jnp.int32)[:, None], (R, LANES))
    cnt_arr = cnt_arr + jnp.zeros((R, LANES), jnp.int32)
    wgt = jnp.stack([_prep_rel(edge_attr_r0[:, 1], 0.0, jnp.float32),
                     _prep_rel(edge_attr_r1[:, 1], 0.0, jnp.float32)]
                    ).reshape(R, NS, NCHUNK * CH)

    h = _tc_proj(x, proj_W, proj_b.reshape(1, H), proj_ln_g.reshape(1, H),
                 proj_ln_b.reshape(1, H))
    base = _tc_base(edge_emb, emlp_W[:EDIM], emlp_b.reshape(1, H))
    w3 = _tc_wcomb(conv_comp, conv_basis.reshape(L, B, H * H)
                   ).reshape(L, R, H, H)

    enh = _sc_enh(base, emlp_W[EDIM], wgt, dst)
    cnt_full = _sc_counts(dst)
    cnt = cnt_full[:, :, 0:1]

    for l in range(L):
        s_lo, s_hi = _sc_seg_both(h, src_m, dst_m, cnt_arr)
        h = _tc_layer(l, h, s_lo, s_hi, enh, cnt, w3, conv_root,
                      conv_bias.reshape(L, 1, H), norm_g.reshape(L, 1, H),
                      norm_b.reshape(L, 1, H))
    return h


# speedup vs baseline: 3.3098x; 1.0002x over previous
import functools

import jax
import jax.numpy as jnp
from jax import lax
from jax.experimental import pallas as pl
from jax.experimental.pallas import tpu as pltpu, tpu_sc as plsc

N = 10000
E = 160000
D_IN = 128
H = 128
R = 2
B = 8
EDIM = 16
L = 3

NC, NS, LANES = 2, 16, 16
CH = 128
NCHUNK = 80
EPT = NCHUNK * CH
EPAD = NS * EPT
NPAD = 10240
RPT = NPAD // NS
HE = H + LANES

_sc_mesh = plsc.VectorSubcoreMesh(core_axis_name="c", subcore_axis_name="s")


@functools.partial(
    pl.kernel,
    out_type=jax.ShapeDtypeStruct((R, NPAD, H), jnp.float32),
    mesh=_sc_mesh,
    scratch_types=[
        pltpu.VMEM((NCHUNK, CH), jnp.int32),
        pltpu.VMEM((NCHUNK * CH,), jnp.float32),
        pltpu.VMEM((CH, H), jnp.float32),
        pltpu.VMEM((H,), jnp.float32),
        pltpu.VMEM((H,), jnp.float32),
        pltpu.VMEM_SHARED((NPAD, H), jnp.float32),
        pltpu.SemaphoreType.DMA,
    ],
)
def _sc_enh(base_hbm, wvec_hbm, w_hbm, dst_hbm, enh_hbm,
            dst_v, w_v, rows_v, base_v, wvec_v, acc_sh, sem):
    c = lax.axis_index("c")
    s = lax.axis_index("s")
    zero16 = jnp.full((LANES,), 0.0, jnp.float32)

    def zero_body(e, carry):
        for g in range(H // LANES):
            rows_v[e, pl.ds(g * LANES, LANES)] = zero16
        return carry

    lax.fori_loop(0, CH, zero_body, 0)
    for q in range(RPT // CH):
        pltpu.sync_copy(rows_v, acc_sh.at[pl.ds(s * RPT + q * CH, CH)])
    pltpu.sync_copy(base_hbm.at[c], base_v)
    pltpu.sync_copy(wvec_hbm, wvec_v)
    pltpu.sync_copy(dst_hbm.at[c, s], dst_v)
    pltpu.sync_copy(w_hbm.at[c, s], w_v)

    base_g = [base_v[pl.ds(g * LANES, LANES)] for g in range(H // LANES)]
    wvec_g = [wvec_v[pl.ds(g * LANES, LANES)] for g in range(H // LANES)]
    plsc.subcore_barrier()

    def chunk_body(j, carry):
        jbase = j * CH

        def grp_body(eb, carry2):
            wv16 = w_v[pl.ds(jbase + eb * LANES, LANES)]
            e0 = eb * LANES
            for k in range(LANES):
                wk = jnp.full((LANES,), wv16[k])
                for g in range(H // LANES):
                    rows_v[e0 + k, pl.ds(g * LANES, LANES)] = jnp.maximum(
                        base_g[g] + wk * wvec_g[g], 0.0)
            return carry2

        lax.fori_loop(0, CH // LANES, grp_body, 0)
        pltpu.sync_copy(rows_v, acc_sh.at[dst_v.at[j]], add=True)
        return carry

    lax.fori_loop(0, NCHUNK, chunk_body, 0)
    plsc.subcore_barrier()
    pltpu.sync_copy(acc_sh.at[pl.ds(s * RPT, RPT)],
                    enh_hbm.at[c, pl.ds(s * RPT, RPT)])


@functools.partial(
    pl.kernel,
    out_type=jax.ShapeDtypeStruct((R, NPAD, H), jnp.float32),
    mesh=_sc_mesh,
    scratch_types=[
        pltpu.VMEM((NCHUNK, CH), jnp.int32),
        pltpu.VMEM((CH, H), jnp.float32),
        pltpu.VMEM_SHARED((NPAD, H), jnp.float32),
        pltpu.SemaphoreType.DMA,
    ],
)
def _sc_counts(dst_hbm, cnt_hbm, dst_v, ones_v, acc_sh, sem):
    c = lax.axis_index("c")
    s = lax.axis_index("s")
    zero16 = jnp.full((LANES,), 0.0, jnp.float32)
    one0 = jnp.where(lax.iota(jnp.int32, LANES) == 0,
                     jnp.full((LANES,), 1.0, jnp.float32),
                     zero16)

    def zero_body(e, carry):
        for g in range(H // LANES):
            ones_v[e, pl.ds(g * LANES, LANES)] = zero16
        return carry

    lax.fori_loop(0, CH, zero_body, 0)
    for q in range(RPT // CH):
        pltpu.sync_copy(ones_v, acc_sh.at[pl.ds(s * RPT + q * CH, CH)])

    def ones_body(e, carry):
        ones_v[e, pl.ds(0, LANES)] = one0
        return carry

    lax.fori_loop(0, CH, ones_body, 0)
    pltpu.sync_copy(dst_hbm.at[c, s], dst_v)
    plsc.subcore_barrier()

    def cnt_body(j, carry):
        pltpu.sync_copy(ones_v, acc_sh.at[dst_v.at[j]], add=True)
        return carry

    lax.fori_loop(0, NCHUNK, cnt_body, 0)
    plsc.subcore_barrier()
    pltpu.sync_copy(acc_sh.at[pl.ds(s * RPT, RPT)],
                    cnt_hbm.at[c, pl.ds(s * RPT, RPT)])


MGCH = 32
MNBUF = 2
MNGRP = EPT // (MNBUF * MGCH)
NHALF = 5120
ACCR = 5128
ARPT = 320
SCRAPM = 5120


@functools.partial(
    pl.kernel,
    out_type=(jax.ShapeDtypeStruct((R, ACCR, H), jnp.float32),
              jax.ShapeDtypeStruct((R, ACCR, H), jnp.float32)),
    mesh=_sc_mesh,
    scratch_types=[
        pltpu.VMEM((2, MNBUF, MGCH), jnp.int32),
        pltpu.VMEM((2, MNBUF, MGCH), jnp.int32),
        pltpu.VMEM((MNBUF, MGCH), jnp.int32),
        pltpu.VMEM((MGCH,), jnp.int32),
        pltpu.VMEM((LANES,), jnp.int32),
    ] + [pltpu.VMEM((MGCH, H), jnp.float32)] * MNBUF
      + [pltpu.VMEM_SHARED((N, H), jnp.float32)]
      + [pltpu.VMEM_SHARED((ACCR, H), jnp.float32)]
      + [pltpu.SemaphoreType.DMA] * (2 * MNBUF + 4),
)
def _sc_seg_both(h_hbm, src_hbm, dst_hbm, cnt_hbm, lo_hbm, hi_hbm,
                 srcb, dstb, dclamp, scrap_v, cnt_v, *rest):
    rows = rest[:MNBUF]
    mir_sh = rest[MNBUF]
    acc_sh = rest[MNBUF + 1]
    gsem = rest[MNBUF + 2:2 * MNBUF + 2]
    ssem = rest[2 * MNBUF + 2:2 * MNBUF + 4]
    isem = rest[2 * MNBUF + 4:2 * MNBUF + 6]
    jsem = rest[2 * MNBUF + 6:2 * MNBUF + 8]
    outs = (lo_hbm, hi_hbm)
    c = lax.axis_index("c")
    s = lax.axis_index("s")
    zero16 = jnp.full((LANES,), 0.0, jnp.float32)
    scrap16 = jnp.full((LANES,), SCRAPM, jnp.int32)
    half16 = jnp.full((LANES,), NHALF, jnp.int32)

    def zero_body(e, carry):
        for g in range(H // LANES):
            rows[0][e, pl.ds(g * LANES, LANES)] = zero16
        return carry

    for kk in range(MGCH // LANES):
        scrap_v[pl.ds(kk * LANES, LANES)] = scrap16
    @pl.when(s < NS - 1)
    def _():
        pltpu.sync_copy(h_hbm.at[pl.ds(s * RPT, RPT)],
                        mir_sh.at[pl.ds(s * RPT, RPT)])

    @pl.when(s == NS - 1)
    def _():
        pltpu.sync_copy(h_hbm.at[pl.ds((NS - 1) * RPT, N - (NS - 1) * RPT)],
                        mir_sh.at[pl.ds((NS - 1) * RPT, N - (NS - 1) * RPT)])

    pltpu.sync_copy(cnt_hbm.at[c], cnt_v)
    cnt0 = cnt_v[pl.ds(0, LANES)][0]
    l1 = jnp.clip(cnt0 - s * EPT, 0, EPT)
    grpsz = MNBUF * MGCH

    for k in range(2):
        lax.fori_loop(0, MGCH, zero_body, 0)
        for q in range(ARPT // MGCH):
            pltpu.sync_copy(rows[0],
                            acc_sh.at[pl.ds(s * ARPT + q * MGCH, MGCH)])

        @pl.when(s == NS - 1)
        def _():
            pltpu.sync_copy(rows[0].at[pl.ds(0, ACCR - NHALF)],
                            acc_sh.at[pl.ds(NHALF, ACCR - NHALF)])

        if k == 0:
            glo = jnp.int32(0)
            ghi = (l1 + grpsz - 1) // grpsz
        else:
            glo = l1 // grpsz
            ghi = jnp.int32(MNGRP)
        n_pairs = (ghi - glo + 1) // 2
        for p in range(2):
            @pl.when(glo + p < ghi)
            def _(p=p, glo=glo, ghi=ghi):
                pltpu.async_copy(src_hbm.at[c, s, glo + p], srcb.at[p],
                                 isem[p])
                pltpu.async_copy(dst_hbm.at[c, s, glo + p], dstb.at[p],
                                 jsem[p])
        plsc.subcore_barrier()
        pltpu.async_copy(rows[0], acc_sh.at[scrap_v], ssem[0], add=True)
        pltpu.async_copy(rows[1], acc_sh.at[scrap_v], ssem[1], add=True)

        def body(jj, carry, k=k, glo=glo, ghi=ghi):
            for p in range(2):
                g = glo + jj * 2 + p

                @pl.when(g < ghi)
                def _(p=p, g=g):
                    pltpu.make_async_copy(src_hbm.at[c, s, g], srcb.at[p],
                                          isem[p]).wait()
                    pltpu.make_async_copy(dst_hbm.at[c, s, g], dstb.at[p],
                                          jsem[p]).wait()
                    gd = [pltpu.async_copy(mir_sh.at[srcb.at[p, b]], rows[b],
                                           gsem[b]) for b in range(MNBUF)]
                    for b in range(MNBUF):
                        gd[b].wait()
                        pltpu.make_async_copy(rows[b], acc_sh.at[scrap_v],
                                              ssem[b]).wait()
                        for v in range(MGCH // LANES):
                            dv = dstb[p, b, pl.ds(v * LANES, LANES)]
                            if k == 0:
                                d = jnp.where(dv < half16, dv, scrap16)
                            else:
                                d = jnp.where(dv >= half16, dv - half16,
                                              scrap16)
                            dclamp[b, pl.ds(v * LANES, LANES)] = d
                        pltpu.async_copy(rows[b], acc_sh.at[dclamp.at[b]],
                                         ssem[b], add=True)
                    nxt = g + 2

                    @pl.when(nxt < ghi)
                    def _():
                        pltpu.async_copy(src_hbm.at[c, s, nxt], srcb.at[p],
                                         isem[p])
                        pltpu.async_copy(dst_hbm.at[c, s, nxt], dstb.at[p],
                                         jsem[p])
            return carry

        lax.fori_loop(0, n_pairs, body, 0)
        for b in range(MNBUF):
            pltpu.make_async_copy(rows[b], acc_sh.at[scrap_v],
                                  ssem[b]).wait()
        plsc.subcore_barrier()
        pltpu.sync_copy(acc_sh.at[pl.ds(s * ARPT, ARPT)],
                        outs[k].at[c, pl.ds(s * ARPT, ARPT)])

        @pl.when(s == NS - 1)
        def _(k=k):
            pltpu.sync_copy(acc_sh.at[pl.ds(NHALF, ACCR - NHALF)],
                            outs[k].at[c, pl.ds(NHALF, ACCR - NHALF)])


def _ln(y, g, b):
    mu = jnp.mean(y, axis=-1, keepdims=True)
    var = jnp.mean((y - mu) ** 2, axis=-1, keepdims=True)
    return (y - mu) * lax.rsqrt(var + 1e-5) * g + b


def _proj_body(x_ref, w_ref, b_ref, g_ref, bb_ref, out_ref):
    y = jnp.dot(x_ref[...], w_ref[...], preferred_element_type=jnp.float32)
    y = y + b_ref[...]
    out_ref[...] = jnp.maximum(_ln(y, g_ref[...], bb_ref[...]), 0.0)


def _base_body(emb_ref, w_ref, b_ref, out_ref):
    out_ref[...] = jnp.dot(emb_ref[...], w_ref[...],
                           preferred_element_type=jnp.float32) + b_ref[...]


def _wcomb_body(comp_ref, basis_ref, out_ref):
    out_ref[0] = jnp.dot(comp_ref[0], basis_ref[0],
                         preferred_element_type=jnp.float32)


def _layer_body(h_ref, slo_ref, shi_ref, enh_ref, cnt_ref, w_ref, root_ref,
                bias_ref, g_ref, b_ref, out_ref):
    c0 = cnt_ref[0]
    c1 = cnt_ref[1]
    deg = jnp.maximum(c0 + c1, 1.0)
    es = 0.1 * (enh_ref[0] + enh_ref[1]) / deg
    use_lo = pl.program_id(0) < NHALF // _ROWB
    s0 = jnp.where(use_lo, slo_ref[0], shi_ref[0])
    s1 = jnp.where(use_lo, slo_ref[1], shi_ref[1])
    m0 = s0 / jnp.maximum(c0, 1.0)
    m1 = s1 / jnp.maximum(c1, 1.0)
    h = h_ref[...]
    agg = (jnp.dot(m0, w_ref[0, 0], preferred_element_type=jnp.float32)
           + jnp.dot(m1, w_ref[0, 1], preferred_element_type=jnp.float32)
           + jnp.dot(h, root_ref[0], preferred_element_type=jnp.float32)
           + bias_ref[0] + es)
    out_ref[...] = jnp.maximum(_ln(agg, g_ref[0], b_ref[0]), 0.0) + h


_ROWB = 1024
_NBLK = (N + _ROWB - 1) // _ROWB


def _tc_proj(x, w, b, g, bb):
    return pl.pallas_call(
        _proj_body,
        grid=(_NBLK,),
        in_specs=[
            pl.BlockSpec((_ROWB, D_IN), lambda i: (i, 0)),
            pl.BlockSpec((D_IN, H), lambda i: (0, 0)),
            pl.BlockSpec((1, H), lambda i: (0, 0)),
            pl.BlockSpec((1, H), lambda i: (0, 0)),
            pl.BlockSpec((1, H), lambda i: (0, 0)),
        ],
        out_specs=pl.BlockSpec((_ROWB, H), lambda i: (i, 0)),
        out_shape=jax.ShapeDtypeStruct((N, H), jnp.float32),
    )(x, w, b, g, bb)


def _tc_base(emb, w16, b):
    return pl.pallas_call(
        _base_body,
        out_shape=jax.ShapeDtypeStruct((R, H), jnp.float32),
    )(emb, w16, b)


def _tc_wcomb(comp, basis_r):
    return pl.pallas_call(
        _wcomb_body,
        grid=(L,),
        in_specs=[
            pl.BlockSpec((1, R, B), lambda l: (l, 0, 0)),
            pl.BlockSpec((1, B, H * H), lambda l: (l, 0, 0)),
        ],
        out_specs=pl.BlockSpec((1, R, H * H), lambda l: (l, 0, 0)),
        out_shape=jax.ShapeDtypeStruct((L, R, H * H), jnp.float32),
    )(comp, basis_r)


def _tc_layer(l, h, s_lo, s_hi, enh, cnt, w3, root, bias, ng, nb):
    nlo = NHALF // _ROWB
    return pl.pallas_call(
        functools.partial(_layer_body),
        grid=(_NBLK,),
        in_specs=[
            pl.BlockSpec((_ROWB, H), lambda i: (i, 0)),
            pl.BlockSpec((R, _ROWB, H),
                         lambda i, _n=nlo: (0, jnp.minimum(i, _n - 1), 0)),
            pl.BlockSpec((R, _ROWB, H),
                         lambda i, _n=nlo: (0, jnp.maximum(i - _n, 0), 0)),
            pl.BlockSpec((R, _ROWB, H), lambda i: (0, i, 0)),
            pl.BlockSpec((R, _ROWB, 1), lambda i: (0, i, 0)),
            pl.BlockSpec((1, R, H, H), lambda i, _l=l: (_l, 0, 0, 0)),
            pl.BlockSpec((1, H, H), lambda i, _l=l: (_l, 0, 0)),
            pl.BlockSpec((1, 1, H), lambda i, _l=l: (_l, 0, 0)),
            pl.BlockSpec((1, 1, H), lambda i, _l=l: (_l, 0, 0)),
            pl.BlockSpec((1, 1, H), lambda i, _l=l: (_l, 0, 0)),
        ],
        out_specs=pl.BlockSpec((_ROWB, H), lambda i: (i, 0)),
        out_shape=jax.ShapeDtypeStruct((N, H), jnp.float32),
    )(h, s_lo, s_hi, enh, cnt, w3, root, bias, ng, nb)


def _partition_rel(src_r, dst_r):
    src_r = src_r.astype(jnp.int32)
    dst_r = dst_r.astype(jnp.int32)
    half = (dst_r >= NHALF).astype(jnp.int32)
    cnt0 = E - half.sum()
    packed = (half << 28) | (src_r << 14) | dst_r
    packed = jax.lax.sort(packed, is_stable=False)
    dp = packed & jnp.int32(16383)
    sp = (packed >> 14) & jnp.int32(16383)
    pad = EPAD - E
    sp = jnp.concatenate([sp, jnp.zeros((pad,), jnp.int32)])
    dp = jnp.concatenate([dp, jnp.full((pad,), 2 * NHALF, jnp.int32)])
    return sp, dp, cnt0


def _prep_rel(idx_row, pad_val, dtype):
    pad = EPAD - E
    arr = jnp.concatenate(
        [idx_row.astype(dtype), jnp.full((pad,), pad_val, dtype)])
    return arr.reshape(NS, NCHUNK, CH)


def kernel(x, edge_index_r0, edge_index_r1, edge_attr_r0, edge_attr_r1,
           proj_W, proj_b, proj_ln_g, proj_ln_b, edge_emb, emlp_W, emlp_b,
           conv_comp, conv_basis, conv_root, conv_bias, norm_g, norm_b):
    src = jnp.stack([_prep_rel(edge_index_r0[0], 0, jnp.int32),
                     _prep_rel(edge_index_r1[0], 0, jnp.int32)])
    dst = jnp.stack([_prep_rel(edge_index_r0[1], N, jnp.int32),
                     _prep_rel(edge_index_r1[1], N, jnp.int32)])
    sp0, dp0, cnt0_0 = _partition_rel(edge_index_r0[0], edge_index_r0[1])
    sp1, dp1, cnt0_1 = _partition_rel(edge_index_r1[0], edge_index_r1[1])
    src_m = jnp.stack([sp0, sp1]).reshape(R, NS, MNGRP, MNBUF, MGCH)
    dst_m = jnp.stack([dp0, dp1]).reshape(R, NS, MNGRP, MNBUF, MGCH)
    cnt_arr = jnp.broadcast_to(
        jnp.stack([cnt0_0, cnt0_1]).astype(jnp.int32)[:, None], (R, LANES))
    cnt_arr = cnt_arr + jnp.zeros((R, LANES), jnp.int32)
    wgt = jnp.stack([_prep_rel(edge_attr_r0[:, 1], 0.0, jnp.float32),
                     _prep_rel(edge_attr_r1[:, 1], 0.0, jnp.float32)]
                    ).reshape(R, NS, NCHUNK * CH)

    h = _tc_proj(x, proj_W, proj_b.reshape(1, H), proj_ln_g.reshape(1, H),
                 proj_ln_b.reshape(1, H))
    base = _tc_base(edge_emb, emlp_W[:EDIM], emlp_b.reshape(1, H))
    w3 = _tc_wcomb(conv_comp, conv_basis.reshape(L, B, H * H)
                   ).reshape(L, R, H, H)

    enh = _sc_enh(base, emlp_W[EDIM], wgt, dst)
    cnt_full = _sc_counts(dst)
    cnt = cnt_full[:, :, 0:1]

    for l in range(L):
        s_lo, s_hi = _sc_seg_both(h, src_m, dst_m, cnt_arr)
        h = _tc_layer(l, h, s_lo, s_hi, enh, cnt, w3, conv_root,
                      conv_bias.reshape(L, 1, H), norm_g.reshape(L, 1, H),
                      norm_b.reshape(L, 1, H))
    return h
